# baseline (speedup 1.0000x reference)
import jax
import jax.numpy as jnp
from jax.experimental import pallas as pl

_YEAR, _MONTH, _DAY, _DOW = 0, 1, 2, 3
_EPS = 0.001
_E = 18
_C = 32
_F = 64

_SBLK = 512
_LBLK = 128


def _stats_body(hvt_ref, out_ref):
    x = hvt_ref[...]
    n = x.shape[0]
    lanes = x.shape[1]
    nbits = 10
    for k in range(1, nbits + 1):
        for j in range(k - 1, -1, -1):
            d = 1 << j
            g = n // (2 * d)
            xr = x.reshape(g, 2, d, lanes)
            a = xr[:, 0]
            b = xr[:, 1]
            lo = jnp.minimum(a, b)
            hi = jnp.maximum(a, b)
            if k == nbits:
                na, nb = lo, hi
            else:
                gi = jax.lax.broadcasted_iota(jnp.int32, (g, 1, 1), 0)
                desc = ((gi >> (k - 1 - j)) & 1) == 1
                na = jnp.where(desc, hi, lo)
                nb = jnp.where(desc, lo, hi)
            x = jnp.stack([na, nb], axis=1).reshape(n, lanes)
    med = 0.5 * (x[511:512, :] + x[512:513, :])
    q1 = 0.25 * x[255:256, :] + 0.75 * x[256:257, :]
    q3 = 0.75 * x[767:768, :] + 0.25 * x[768:769, :]
    iqr = (q3 - q1) + _EPS
    out_ref[0:1, :] = med
    out_ref[1:2, :] = iqr
    out_ref[2:8, :] = jnp.broadcast_to(med, (6, lanes))


_NOH = 7


def _onehot_feats(tf, ref_year):
    rows = tf.shape[0]
    dy = jnp.clip(ref_year - tf[:, _YEAR:_YEAR + 1], 0, 10)
    vals = jnp.concatenate(
        [jnp.broadcast_to(v, (rows, _NOH)) for v in
         (dy, tf[:, _MONTH:_MONTH + 1], tf[:, _DAY:_DAY + 1],
          tf[:, _DOW:_DOW + 1])], axis=1)
    slot = jax.lax.broadcasted_iota(jnp.int32, (rows, 4 * _NOH), 1) % _NOH
    return jnp.where(vals == slot, 1.0, 0.0)


def _pos_embed_block(tf, ref_year, yt_ref, mt_ref, dt_ref, wt_ref):
    year = tf[:, _YEAR:_YEAR + 1]
    month = tf[:, _MONTH:_MONTH + 1]
    day = tf[:, _DAY:_DAY + 1]
    dow = tf[:, _DOW:_DOW + 1]
    dy = jnp.clip(ref_year - year, 0, 10)

    def lookup(idx, table_ref, rows, width):
        acc = jnp.zeros((idx.shape[0], width), jnp.float32)
        for v in range(rows):
            row = table_ref[v:v + 1, :]
            acc = acc + jnp.where(idx == v, 1.0, 0.0) * row
        return acc

    pe_y = lookup(dy, yt_ref, 11, 4)
    pe_m = lookup(month, mt_ref, 12, 4)
    pe_d = lookup(day, dt_ref, 31, 6)
    pe_w = lookup(dow, wt_ref, 7, 4)
    return pe_y, pe_m, pe_d, pe_w


def _expand_body(hv_ref, med_ref, iqr_ref, tf_ref, ttf_ref, tv_ref, tci_ref,
                 ry_ref, m_ref, yt_ref, mt_ref, dt_ref, wt_ref,
                 out1_ref, out2_ref, out3_ref):
    s = pl.program_id(1)
    ref_year = ry_ref[0, 0, 0]
    med = med_ref[0]
    iqr = iqr_ref[0]
    m = m_ref[...]

    hv = hv_ref[0]
    hs = (hv - med) / iqr
    rows = hs.shape[0]
    oh = _onehot_feats(tf_ref[0], ref_year)
    feats = jnp.concatenate(
        [hs, oh, jnp.ones((rows, 1), jnp.float32),
         jnp.zeros((rows, _F - _C - 4 * _NOH - 1), jnp.float32)], axis=-1)
    out1_ref[0] = jax.lax.dot(
        feats, m, precision=jax.lax.Precision.HIGHEST,
        preferred_element_type=jnp.float32)

    @pl.when(s == 0)
    def _targets():
        toh = _onehot_feats(ttf_ref[0], ref_year)
        p = toh.shape[0]
        tfeats = jnp.concatenate(
            [jnp.zeros((p, _C), jnp.float32), toh,
             jnp.zeros((p, _F - _C - 4 * _NOH), jnp.float32)], axis=-1)
        out2_ref[0] = jax.lax.dot(
            tfeats, m, precision=jax.lax.Precision.HIGHEST,
            preferred_element_type=jnp.float32)
        tci = tci_ref[0]
        med_g = jnp.zeros(tci.shape, jnp.float32)
        iqr_g = jnp.zeros(tci.shape, jnp.float32)
        for c in range(_C):
            hit = jnp.where(tci == c, 1.0, 0.0)
            med_g = med_g + hit * med[:, c:c + 1]
            iqr_g = iqr_g + hit * iqr[:, c:c + 1]
        out3_ref[0] = (tv_ref[0] - med_g) / iqr_g


def _build_m(w_expand, b_expand, yt, mt, dt, wt):
    ce = jnp.arange(_C * _E)
    cidx = ce // _E
    eidx = ce % _E
    rows = jnp.arange(_F)[:, None]
    m = jnp.where(rows == cidx[None, :], w_expand[eidx][None, :], 0.0)
    for t, (tab, off) in enumerate(((yt, 0), (mt, 4), (dt, 8), (wt, 14))):
        width = tab.shape[1]
        base = _C + t * _NOH
        inseg = (eidx >= off) & (eidx < off + width)
        k = rows - base
        hit = (k >= 0) & (k < _NOH) & inseg[None, :]
        gathered = tab[jnp.clip(k, 0, _NOH - 1), jnp.clip(eidx - off, 0, width - 1)[None, :]]
        m = m + jnp.where(hit, gathered, 0.0)
    m = m + jnp.where(rows == _C + 4 * _NOH, b_expand[eidx][None, :], 0.0)
    return m.astype(jnp.float32)


@jax.jit
def kernel(history_values, target_values, target_channels_indices,
           history_time_features, target_time_features,
           pos_year_table, pos_month_table, pos_day_table, pos_dow_table,
           W_expand, b_expand):
    B, S, C = history_values.shape
    P, T = target_values.shape[1], target_values.shape[2]

    hvt = jnp.transpose(history_values, (1, 0, 2)).reshape(S, B * C)
    nlb = (B * C) // _LBLK
    stats = pl.pallas_call(
        _stats_body,
        grid=(nlb,),
        in_specs=[pl.BlockSpec((S, _LBLK), lambda i: (0, i))],
        out_specs=pl.BlockSpec((8, _LBLK), lambda i: (0, i)),
        out_shape=jax.ShapeDtypeStruct((8, B * C), jnp.float32),
    )(hvt)
    med_bc = stats[0].reshape(B, 1, C)
    iqr_bc = stats[1].reshape(B, 1, C)

    m = _build_m(W_expand, b_expand, pos_year_table, pos_month_table,
                 pos_day_table, pos_dow_table)
    ref_year = history_time_features[:, S - 1:S, _YEAR:_YEAR + 1]
    tci3 = target_channels_indices.reshape(B, 1, T)
    nsb = S // _SBLK
    out1, out2, out3 = pl.pallas_call(
        _expand_body,
        grid=(B, nsb),
        in_specs=[
            pl.BlockSpec((1, _SBLK, C), lambda b, s: (b, s, 0)),
            pl.BlockSpec((1, 1, C), lambda b, s: (b, 0, 0)),
            pl.BlockSpec((1, 1, C), lambda b, s: (b, 0, 0)),
            pl.BlockSpec((1, _SBLK, 4), lambda b, s: (b, s, 0)),
            pl.BlockSpec((1, P, 4), lambda b, s: (b, 0, 0)),
            pl.BlockSpec((1, P, T), lambda b, s: (b, 0, 0)),
            pl.BlockSpec((1, 1, T), lambda b, s: (b, 0, 0)),
            pl.BlockSpec((1, 1, 1), lambda b, s: (b, 0, 0)),
            pl.BlockSpec((_F, C * _E), lambda b, s: (0, 0)),
            pl.BlockSpec((11, 4), lambda b, s: (0, 0)),
            pl.BlockSpec((12, 4), lambda b, s: (0, 0)),
            pl.BlockSpec((31, 6), lambda b, s: (0, 0)),
            pl.BlockSpec((7, 4), lambda b, s: (0, 0)),
        ],
        out_specs=[
            pl.BlockSpec((1, _SBLK, C * _E), lambda b, s: (b, s, 0)),
            pl.BlockSpec((1, P, C * _E), lambda b, s: (b, 0, 0)),
            pl.BlockSpec((1, P, T), lambda b, s: (b, 0, 0)),
        ],
        out_shape=[
            jax.ShapeDtypeStruct((B, S, C * _E), jnp.float32),
            jax.ShapeDtypeStruct((B, P, C * _E), jnp.float32),
            jax.ShapeDtypeStruct((B, P, T), jnp.float32),
        ],
    )(history_values, med_bc, iqr_bc, history_time_features,
      target_time_features, target_values, tci3, ref_year, m,
      pos_year_table, pos_month_table, pos_day_table, pos_dow_table)

    return out1, out2.reshape(B, P, C, _E), out3


# speedup vs baseline: 2.6645x; 2.6645x over previous
import jax
import jax.numpy as jnp
from jax.experimental import pallas as pl

_YEAR, _MONTH, _DAY, _DOW = 0, 1, 2, 3
_EPS = 0.001
_E = 18
_C = 32
_F = 64

_SBLK = 512
_LBLK = 128


def _stats_body(hvt_ref, out_ref):
    x = hvt_ref[...]
    n = x.shape[0]
    lanes = x.shape[1]
    nbits = 10
    for k in range(1, nbits + 1):
        for j in range(k - 1, -1, -1):
            d = 1 << j
            g = n // (2 * d)
            xr = x.reshape(g, 2, d, lanes)
            a = xr[:, 0]
            b = xr[:, 1]
            lo = jnp.minimum(a, b)
            hi = jnp.maximum(a, b)
            if k == nbits:
                na, nb = lo, hi
            else:
                gi = jax.lax.broadcasted_iota(jnp.int32, (g, 1, 1), 0)
                desc = ((gi >> (k - 1 - j)) & 1) == 1
                na = jnp.where(desc, hi, lo)
                nb = jnp.where(desc, lo, hi)
            x = jnp.stack([na, nb], axis=1).reshape(n, lanes)
    med = 0.5 * (x[511:512, :] + x[512:513, :])
    q1 = 0.25 * x[255:256, :] + 0.75 * x[256:257, :]
    q3 = 0.75 * x[767:768, :] + 0.25 * x[768:769, :]
    iqr = (q3 - q1) + _EPS
    out_ref[0:1, :] = med
    out_ref[1:2, :] = iqr
    out_ref[2:8, :] = jnp.broadcast_to(med, (6, lanes))


_NOH = 7


def _onehot_feats(tf, ref_year):
    rows = tf.shape[0]
    dy = jnp.clip(ref_year - tf[:, _YEAR:_YEAR + 1], 0, 10)
    vals = jnp.concatenate(
        [jnp.broadcast_to(v, (rows, _NOH)) for v in
         (dy, tf[:, _MONTH:_MONTH + 1], tf[:, _DAY:_DAY + 1],
          tf[:, _DOW:_DOW + 1])], axis=1)
    slot = jax.lax.broadcasted_iota(jnp.int32, (rows, 4 * _NOH), 1) % _NOH
    return jnp.where(vals == slot, 1.0, 0.0)


def _pos_embed_block(tf, ref_year, yt_ref, mt_ref, dt_ref, wt_ref):
    year = tf[:, _YEAR:_YEAR + 1]
    month = tf[:, _MONTH:_MONTH + 1]
    day = tf[:, _DAY:_DAY + 1]
    dow = tf[:, _DOW:_DOW + 1]
    dy = jnp.clip(ref_year - year, 0, 10)

    def lookup(idx, table_ref, rows, width):
        acc = jnp.zeros((idx.shape[0], width), jnp.float32)
        for v in range(rows):
            row = table_ref[v:v + 1, :]
            acc = acc + jnp.where(idx == v, 1.0, 0.0) * row
        return acc

    pe_y = lookup(dy, yt_ref, 11, 4)
    pe_m = lookup(month, mt_ref, 12, 4)
    pe_d = lookup(day, dt_ref, 31, 6)
    pe_w = lookup(dow, wt_ref, 7, 4)
    return pe_y, pe_m, pe_d, pe_w


def _expand_body(hv_ref, med_ref, iqr_ref, tf_ref, ttf_ref, tv_ref, tci_ref,
                 ry_ref, m_ref, yt_ref, mt_ref, dt_ref, wt_ref,
                 out1_ref, out2_ref, out3_ref):
    s = pl.program_id(1)
    ref_year = ry_ref[0, 0, 0]
    med = med_ref[0]
    iqr = iqr_ref[0]
    m = m_ref[...]

    hv = hv_ref[0]
    hs = (hv - med) / iqr
    rows = hs.shape[0]
    oh = _onehot_feats(tf_ref[0], ref_year)
    feats = jnp.concatenate(
        [hs, oh, jnp.ones((rows, 1), jnp.float32),
         jnp.zeros((rows, _F - _C - 4 * _NOH - 1), jnp.float32)], axis=-1)
    out1_ref[0] = jax.lax.dot(
        feats, m, precision=jax.lax.Precision.HIGHEST,
        preferred_element_type=jnp.float32)

    @pl.when(s == 0)
    def _targets():
        toh = _onehot_feats(ttf_ref[0], ref_year)
        p = toh.shape[0]
        tfeats = jnp.concatenate(
            [jnp.zeros((p, _C), jnp.float32), toh,
             jnp.zeros((p, _F - _C - 4 * _NOH), jnp.float32)], axis=-1)
        out2_ref[0] = jax.lax.dot(
            tfeats, m, precision=jax.lax.Precision.HIGHEST,
            preferred_element_type=jnp.float32)
        tci = tci_ref[0]
        med_g = jnp.zeros(tci.shape, jnp.float32)
        iqr_g = jnp.zeros(tci.shape, jnp.float32)
        for c in range(_C):
            hit = jnp.where(tci == c, 1.0, 0.0)
            med_g = med_g + hit * med[:, c:c + 1]
            iqr_g = iqr_g + hit * iqr[:, c:c + 1]
        out3_ref[0] = (tv_ref[0] - med_g) / iqr_g


def _build_m(w_expand, b_expand, yt, mt, dt, wt):
    ce = jnp.arange(_C * _E)
    cidx = ce // _E
    rows32 = jnp.arange(_C)[:, None]
    wfull = jnp.tile(w_expand, _C)[None, :]
    m_w = jnp.where(rows32 == cidx[None, :], wfull, 0.0)
    blocks = [m_w]
    for tab, off in ((yt, 0), (mt, 4), (dt, 8), (wt, 14)):
        width = tab.shape[1]
        padded = jnp.pad(tab[:_NOH], ((0, 0), (off, _E - off - width)))
        blocks.append(jnp.tile(padded, (1, _C)))
    blocks.append(jnp.tile(b_expand, _C)[None, :])
    blocks.append(jnp.zeros((_F - _C - 4 * _NOH - 1, _C * _E)))
    return jnp.concatenate(blocks, axis=0).astype(jnp.float32)


@jax.jit
def kernel(history_values, target_values, target_channels_indices,
           history_time_features, target_time_features,
           pos_year_table, pos_month_table, pos_day_table, pos_dow_table,
           W_expand, b_expand):
    B, S, C = history_values.shape
    P, T = target_values.shape[1], target_values.shape[2]

    hvt = jnp.transpose(history_values, (1, 0, 2)).reshape(S, B * C)
    nlb = (B * C) // _LBLK
    stats = pl.pallas_call(
        _stats_body,
        grid=(nlb,),
        in_specs=[pl.BlockSpec((S, _LBLK), lambda i: (0, i))],
        out_specs=pl.BlockSpec((8, _LBLK), lambda i: (0, i)),
        out_shape=jax.ShapeDtypeStruct((8, B * C), jnp.float32),
    )(hvt)
    med_bc = stats[0].reshape(B, 1, C)
    iqr_bc = stats[1].reshape(B, 1, C)

    m = _build_m(W_expand, b_expand, pos_year_table, pos_month_table,
                 pos_day_table, pos_dow_table)
    ref_year = history_time_features[:, S - 1:S, _YEAR:_YEAR + 1]
    tci3 = target_channels_indices.reshape(B, 1, T)
    nsb = S // _SBLK
    out1, out2, out3 = pl.pallas_call(
        _expand_body,
        grid=(B, nsb),
        in_specs=[
            pl.BlockSpec((1, _SBLK, C), lambda b, s: (b, s, 0)),
            pl.BlockSpec((1, 1, C), lambda b, s: (b, 0, 0)),
            pl.BlockSpec((1, 1, C), lambda b, s: (b, 0, 0)),
            pl.BlockSpec((1, _SBLK, 4), lambda b, s: (b, s, 0)),
            pl.BlockSpec((1, P, 4), lambda b, s: (b, 0, 0)),
            pl.BlockSpec((1, P, T), lambda b, s: (b, 0, 0)),
            pl.BlockSpec((1, 1, T), lambda b, s: (b, 0, 0)),
            pl.BlockSpec((1, 1, 1), lambda b, s: (b, 0, 0)),
            pl.BlockSpec((_F, C * _E), lambda b, s: (0, 0)),
            pl.BlockSpec((11, 4), lambda b, s: (0, 0)),
            pl.BlockSpec((12, 4), lambda b, s: (0, 0)),
            pl.BlockSpec((31, 6), lambda b, s: (0, 0)),
            pl.BlockSpec((7, 4), lambda b, s: (0, 0)),
        ],
        out_specs=[
            pl.BlockSpec((1, _SBLK, C * _E), lambda b, s: (b, s, 0)),
            pl.BlockSpec((1, P, C * _E), lambda b, s: (b, 0, 0)),
            pl.BlockSpec((1, P, T), lambda b, s: (b, 0, 0)),
        ],
        out_shape=[
            jax.ShapeDtypeStruct((B, S, C * _E), jnp.float32),
            jax.ShapeDtypeStruct((B, P, C * _E), jnp.float32),
            jax.ShapeDtypeStruct((B, P, T), jnp.float32),
        ],
    )(history_values, med_bc, iqr_bc, history_time_features,
      target_time_features, target_values, tci3, ref_year, m,
      pos_year_table, pos_month_table, pos_day_table, pos_dow_table)

    return out1, out2.reshape(B, P, C, _E), out3


# speedup vs baseline: 2.8012x; 1.0513x over previous
import jax
import jax.numpy as jnp
from jax.experimental import pallas as pl

_YEAR, _MONTH, _DAY, _DOW = 0, 1, 2, 3
_EPS = 0.001
_E = 18
_C = 32
_F = 64

_SBLK = 1024
_LBLK = 128


def _stats_body(hvt_ref, out_ref):
    x = hvt_ref[...]
    n = x.shape[0]
    lanes = x.shape[1]
    nbits = 10
    for k in range(1, nbits + 1):
        for j in range(k - 1, -1, -1):
            d = 1 << j
            g = n // (2 * d)
            xr = x.reshape(g, 2, d, lanes)
            a = xr[:, 0]
            b = xr[:, 1]
            lo = jnp.minimum(a, b)
            hi = jnp.maximum(a, b)
            if k == nbits:
                na, nb = lo, hi
            else:
                gi = jax.lax.broadcasted_iota(jnp.int32, (g, 1, 1), 0)
                desc = ((gi >> (k - 1 - j)) & 1) == 1
                na = jnp.where(desc, hi, lo)
                nb = jnp.where(desc, lo, hi)
            x = jnp.stack([na, nb], axis=1).reshape(n, lanes)
    med = 0.5 * (x[511:512, :] + x[512:513, :])
    q1 = 0.25 * x[255:256, :] + 0.75 * x[256:257, :]
    q3 = 0.75 * x[767:768, :] + 0.25 * x[768:769, :]
    iqr = (q3 - q1) + _EPS
    out_ref[0:1, :] = med
    out_ref[1:2, :] = iqr
    out_ref[2:8, :] = jnp.broadcast_to(med, (6, lanes))


_NOH = 7


def _onehot_feats(tf, ref_year):
    rows = tf.shape[0]
    dy = jnp.clip(ref_year - tf[:, _YEAR:_YEAR + 1], 0, 10)
    vals = jnp.concatenate(
        [jnp.broadcast_to(v, (rows, _NOH)) for v in
         (dy, tf[:, _MONTH:_MONTH + 1], tf[:, _DAY:_DAY + 1],
          tf[:, _DOW:_DOW + 1])], axis=1)
    slot = jax.lax.broadcasted_iota(jnp.int32, (rows, 4 * _NOH), 1) % _NOH
    return jnp.where(vals == slot, 1.0, 0.0)


def _pos_embed_block(tf, ref_year, yt_ref, mt_ref, dt_ref, wt_ref):
    year = tf[:, _YEAR:_YEAR + 1]
    month = tf[:, _MONTH:_MONTH + 1]
    day = tf[:, _DAY:_DAY + 1]
    dow = tf[:, _DOW:_DOW + 1]
    dy = jnp.clip(ref_year - year, 0, 10)

    def lookup(idx, table_ref, rows, width):
        acc = jnp.zeros((idx.shape[0], width), jnp.float32)
        for v in range(rows):
            row = table_ref[v:v + 1, :]
            acc = acc + jnp.where(idx == v, 1.0, 0.0) * row
        return acc

    pe_y = lookup(dy, yt_ref, 11, 4)
    pe_m = lookup(month, mt_ref, 12, 4)
    pe_d = lookup(day, dt_ref, 31, 6)
    pe_w = lookup(dow, wt_ref, 7, 4)
    return pe_y, pe_m, pe_d, pe_w


def _expand_body(hv_ref, med_ref, iqr_ref, tf_ref, ttf_ref, tv_ref, tci_ref,
                 ry_ref, m_ref, yt_ref, mt_ref, dt_ref, wt_ref,
                 out1_ref, out2_ref, out3_ref):
    s = pl.program_id(1)
    ref_year = ry_ref[0, 0, 0]
    med = med_ref[0]
    iqr = iqr_ref[0]
    m = m_ref[...]

    hv = hv_ref[0]
    hs = (hv - med) / iqr
    rows = hs.shape[0]
    oh = _onehot_feats(tf_ref[0], ref_year)
    feats = jnp.concatenate(
        [hs, oh, jnp.ones((rows, 1), jnp.float32),
         jnp.zeros((rows, _F - _C - 4 * _NOH - 1), jnp.float32)], axis=-1)
    out1_ref[0] = jax.lax.dot(
        feats, m, precision=jax.lax.Precision.HIGHEST,
        preferred_element_type=jnp.float32)

    @pl.when(s == 0)
    def _targets():
        toh = _onehot_feats(ttf_ref[0], ref_year)
        p = toh.shape[0]
        tfeats = jnp.concatenate(
            [jnp.zeros((p, _C), jnp.float32), toh,
             jnp.zeros((p, _F - _C - 4 * _NOH), jnp.float32)], axis=-1)
        out2_ref[0] = jax.lax.dot(
            tfeats, m, precision=jax.lax.Precision.HIGHEST,
            preferred_element_type=jnp.float32)
        tci = tci_ref[0]
        med_g = jnp.zeros(tci.shape, jnp.float32)
        iqr_g = jnp.zeros(tci.shape, jnp.float32)
        for c in range(_C):
            hit = jnp.where(tci == c, 1.0, 0.0)
            med_g = med_g + hit * med[:, c:c + 1]
            iqr_g = iqr_g + hit * iqr[:, c:c + 1]
        out3_ref[0] = (tv_ref[0] - med_g) / iqr_g


def _build_m(w_expand, b_expand, yt, mt, dt, wt):
    ce = jnp.arange(_C * _E)
    cidx = ce // _E
    rows32 = jnp.arange(_C)[:, None]
    wfull = jnp.tile(w_expand, _C)[None, :]
    m_w = jnp.where(rows32 == cidx[None, :], wfull, 0.0)
    blocks = [m_w]
    for tab, off in ((yt, 0), (mt, 4), (dt, 8), (wt, 14)):
        width = tab.shape[1]
        padded = jnp.pad(tab[:_NOH], ((0, 0), (off, _E - off - width)))
        blocks.append(jnp.tile(padded, (1, _C)))
    blocks.append(jnp.tile(b_expand, _C)[None, :])
    blocks.append(jnp.zeros((_F - _C - 4 * _NOH - 1, _C * _E)))
    return jnp.concatenate(blocks, axis=0).astype(jnp.float32)


@jax.jit
def kernel(history_values, target_values, target_channels_indices,
           history_time_features, target_time_features,
           pos_year_table, pos_month_table, pos_day_table, pos_dow_table,
           W_expand, b_expand):
    B, S, C = history_values.shape
    P, T = target_values.shape[1], target_values.shape[2]

    hvt = jnp.transpose(history_values, (1, 0, 2)).reshape(S, B * C)
    nlb = (B * C) // _LBLK
    stats = pl.pallas_call(
        _stats_body,
        grid=(nlb,),
        in_specs=[pl.BlockSpec((S, _LBLK), lambda i: (0, i))],
        out_specs=pl.BlockSpec((8, _LBLK), lambda i: (0, i)),
        out_shape=jax.ShapeDtypeStruct((8, B * C), jnp.float32),
    )(hvt)
    med_bc = stats[0].reshape(B, 1, C)
    iqr_bc = stats[1].reshape(B, 1, C)

    m = _build_m(W_expand, b_expand, pos_year_table, pos_month_table,
                 pos_day_table, pos_dow_table)
    ref_year = history_time_features[:, S - 1:S, _YEAR:_YEAR + 1]
    tci3 = target_channels_indices.reshape(B, 1, T)
    nsb = S // _SBLK
    out1, out2, out3 = pl.pallas_call(
        _expand_body,
        grid=(B, nsb),
        in_specs=[
            pl.BlockSpec((1, _SBLK, C), lambda b, s: (b, s, 0)),
            pl.BlockSpec((1, 1, C), lambda b, s: (b, 0, 0)),
            pl.BlockSpec((1, 1, C), lambda b, s: (b, 0, 0)),
            pl.BlockSpec((1, _SBLK, 4), lambda b, s: (b, s, 0)),
            pl.BlockSpec((1, P, 4), lambda b, s: (b, 0, 0)),
            pl.BlockSpec((1, P, T), lambda b, s: (b, 0, 0)),
            pl.BlockSpec((1, 1, T), lambda b, s: (b, 0, 0)),
            pl.BlockSpec((1, 1, 1), lambda b, s: (b, 0, 0)),
            pl.BlockSpec((_F, C * _E), lambda b, s: (0, 0)),
            pl.BlockSpec((11, 4), lambda b, s: (0, 0)),
            pl.BlockSpec((12, 4), lambda b, s: (0, 0)),
            pl.BlockSpec((31, 6), lambda b, s: (0, 0)),
            pl.BlockSpec((7, 4), lambda b, s: (0, 0)),
        ],
        out_specs=[
            pl.BlockSpec((1, _SBLK, C * _E), lambda b, s: (b, s, 0)),
            pl.BlockSpec((1, P, C * _E), lambda b, s: (b, 0, 0)),
            pl.BlockSpec((1, P, T), lambda b, s: (b, 0, 0)),
        ],
        out_shape=[
            jax.ShapeDtypeStruct((B, S, C * _E), jnp.float32),
            jax.ShapeDtypeStruct((B, P, C * _E), jnp.float32),
            jax.ShapeDtypeStruct((B, P, T), jnp.float32),
        ],
    )(history_values, med_bc, iqr_bc, history_time_features,
      target_time_features, target_values, tci3, ref_year, m,
      pos_year_table, pos_month_table, pos_day_table, pos_dow_table)

    return out1, out2.reshape(B, P, C, _E), out3


# speedup vs baseline: 3.1009x; 1.1070x over previous
import jax
import jax.numpy as jnp
from jax.experimental import pallas as pl

_YEAR, _MONTH, _DAY, _DOW = 0, 1, 2, 3
_EPS = 0.001
_E = 18
_C = 32
_F = 64

_SBLK = 1024
_LBLK = 128


def _chunk_stage(y, d, k=None, desc=None):
    n, lanes = y.shape
    g = n // (2 * d)
    xr = y.reshape(g, 2, d, lanes)
    a = xr[:, 0]
    b = xr[:, 1]
    lo = jnp.minimum(a, b)
    hi = jnp.maximum(a, b)
    if desc is None:
        gi = jax.lax.broadcasted_iota(jnp.int32, (g, 1, 1), 0)
        dm = ((gi >> (k - 1 - (d.bit_length() - 1))) & 1) == 1
        na = jnp.where(dm, hi, lo)
        nb = jnp.where(dm, lo, hi)
    elif desc:
        na, nb = hi, lo
    else:
        na, nb = lo, hi
    return jnp.stack([na, nb], axis=1).reshape(n, lanes)


def _sort128(y, final_desc):
    for k in range(1, 8):
        for j in range(k - 1, -1, -1):
            d = 1 << j
            if k == 7:
                y = _chunk_stage(y, d, desc=final_desc)
            else:
                y = _chunk_stage(y, d, k=k)
    return y


def _merge128(y, desc):
    for j in range(6, -1, -1):
        y = _chunk_stage(y, 1 << j, desc=desc)
    return y


def _minmax(a, b, desc):
    lo = jnp.minimum(a, b)
    hi = jnp.maximum(a, b)
    return (hi, lo) if desc else (lo, hi)


def _stats_body(hvt_ref, out_ref):
    lanes = hvt_ref.shape[1]
    ch = [hvt_ref[c * 128:(c + 1) * 128, :] for c in range(8)]
    ch = [_sort128(ch[c], final_desc=bool(c & 1)) for c in range(8)]
    for p in range(4):
        desc = bool(p & 1)
        a, b = _minmax(ch[2 * p], ch[2 * p + 1], desc)
        ch[2 * p] = _merge128(a, desc)
        ch[2 * p + 1] = _merge128(b, desc)
    for g2 in range(2):
        desc = bool(g2 & 1)
        base = g2 * 4
        for t in range(2):
            ch[base + t], ch[base + t + 2] = _minmax(ch[base + t], ch[base + t + 2], desc)
        for t in (0, 2):
            ch[base + t], ch[base + t + 1] = _minmax(ch[base + t], ch[base + t + 1], desc)
        for t in range(4):
            ch[base + t] = _merge128(ch[base + t], desc)
    for t in range(4):
        ch[t], ch[t + 4] = _minmax(ch[t], ch[t + 4], False)
    for t in (0, 1, 4, 5):
        ch[t], ch[t + 2] = _minmax(ch[t], ch[t + 2], False)
    for t in (0, 2, 4, 6):
        ch[t], ch[t + 1] = _minmax(ch[t], ch[t + 1], False)
    ch = [_merge128(c, False) for c in ch]
    med = 0.5 * (ch[3][127:128, :] + ch[4][0:1, :])
    q1 = 0.25 * ch[1][127:128, :] + 0.75 * ch[2][0:1, :]
    q3 = 0.75 * ch[5][127:128, :] + 0.25 * ch[6][0:1, :]
    iqr = (q3 - q1) + _EPS
    out_ref[0:1, :] = med
    out_ref[1:2, :] = iqr
    out_ref[2:8, :] = jnp.broadcast_to(med, (6, lanes))


_NOH = 7


def _onehot_feats(tf, ref_year):
    rows = tf.shape[0]
    dy = jnp.clip(ref_year - tf[:, _YEAR:_YEAR + 1], 0, 10)
    vals = jnp.concatenate(
        [jnp.broadcast_to(v, (rows, _NOH)) for v in
         (dy, tf[:, _MONTH:_MONTH + 1], tf[:, _DAY:_DAY + 1],
          tf[:, _DOW:_DOW + 1])], axis=1)
    slot = jax.lax.broadcasted_iota(jnp.int32, (rows, 4 * _NOH), 1) % _NOH
    return jnp.where(vals == slot, 1.0, 0.0)


def _pos_embed_block(tf, ref_year, yt_ref, mt_ref, dt_ref, wt_ref):
    year = tf[:, _YEAR:_YEAR + 1]
    month = tf[:, _MONTH:_MONTH + 1]
    day = tf[:, _DAY:_DAY + 1]
    dow = tf[:, _DOW:_DOW + 1]
    dy = jnp.clip(ref_year - year, 0, 10)

    def lookup(idx, table_ref, rows, width):
        acc = jnp.zeros((idx.shape[0], width), jnp.float32)
        for v in range(rows):
            row = table_ref[v:v + 1, :]
            acc = acc + jnp.where(idx == v, 1.0, 0.0) * row
        return acc

    pe_y = lookup(dy, yt_ref, 11, 4)
    pe_m = lookup(month, mt_ref, 12, 4)
    pe_d = lookup(day, dt_ref, 31, 6)
    pe_w = lookup(dow, wt_ref, 7, 4)
    return pe_y, pe_m, pe_d, pe_w


def _expand_body(hv_ref, med_ref, iqr_ref, tf_ref, ttf_ref, tv_ref, tci_ref,
                 ry_ref, m_ref, yt_ref, mt_ref, dt_ref, wt_ref,
                 out1_ref, out2_ref, out3_ref):
    s = pl.program_id(1)
    ref_year = ry_ref[0, 0, 0]
    med = med_ref[0]
    iqr = iqr_ref[0]
    m = m_ref[...]

    hv = hv_ref[0]
    hs = (hv - med) / iqr
    rows = hs.shape[0]
    oh = _onehot_feats(tf_ref[0], ref_year)
    feats = jnp.concatenate(
        [hs, oh, jnp.ones((rows, 1), jnp.float32),
         jnp.zeros((rows, _F - _C - 4 * _NOH - 1), jnp.float32)], axis=-1)
    out1_ref[0] = jax.lax.dot(
        feats, m, precision=jax.lax.Precision.HIGHEST,
        preferred_element_type=jnp.float32)

    @pl.when(s == 0)
    def _targets():
        toh = _onehot_feats(ttf_ref[0], ref_year)
        p = toh.shape[0]
        tfeats = jnp.concatenate(
            [jnp.zeros((p, _C), jnp.float32), toh,
             jnp.zeros((p, _F - _C - 4 * _NOH), jnp.float32)], axis=-1)
        out2_ref[0] = jax.lax.dot(
            tfeats, m, precision=jax.lax.Precision.HIGHEST,
            preferred_element_type=jnp.float32)
        tci = tci_ref[0]
        med_g = jnp.zeros(tci.shape, jnp.float32)
        iqr_g = jnp.zeros(tci.shape, jnp.float32)
        for c in range(_C):
            hit = jnp.where(tci == c, 1.0, 0.0)
            med_g = med_g + hit * med[:, c:c + 1]
            iqr_g = iqr_g + hit * iqr[:, c:c + 1]
        out3_ref[0] = (tv_ref[0] - med_g) / iqr_g


def _build_m(w_expand, b_expand, yt, mt, dt, wt):
    ce = jnp.arange(_C * _E)
    cidx = ce // _E
    rows32 = jnp.arange(_C)[:, None]
    wfull = jnp.tile(w_expand, _C)[None, :]
    m_w = jnp.where(rows32 == cidx[None, :], wfull, 0.0)
    blocks = [m_w]
    for tab, off in ((yt, 0), (mt, 4), (dt, 8), (wt, 14)):
        width = tab.shape[1]
        padded = jnp.pad(tab[:_NOH], ((0, 0), (off, _E - off - width)))
        blocks.append(jnp.tile(padded, (1, _C)))
    blocks.append(jnp.tile(b_expand, _C)[None, :])
    blocks.append(jnp.zeros((_F - _C - 4 * _NOH - 1, _C * _E)))
    return jnp.concatenate(blocks, axis=0).astype(jnp.float32)


@jax.jit
def kernel(history_values, target_values, target_channels_indices,
           history_time_features, target_time_features,
           pos_year_table, pos_month_table, pos_day_table, pos_dow_table,
           W_expand, b_expand):
    B, S, C = history_values.shape
    P, T = target_values.shape[1], target_values.shape[2]

    hvt = jnp.transpose(history_values, (1, 0, 2)).reshape(S, B * C)
    nlb = (B * C) // _LBLK
    stats = pl.pallas_call(
        _stats_body,
        grid=(nlb,),
        in_specs=[pl.BlockSpec((S, _LBLK), lambda i: (0, i))],
        out_specs=pl.BlockSpec((8, _LBLK), lambda i: (0, i)),
        out_shape=jax.ShapeDtypeStruct((8, B * C), jnp.float32),
    )(hvt)
    med_bc = stats[0].reshape(B, 1, C)
    iqr_bc = stats[1].reshape(B, 1, C)

    m = _build_m(W_expand, b_expand, pos_year_table, pos_month_table,
                 pos_day_table, pos_dow_table)
    ref_year = history_time_features[:, S - 1:S, _YEAR:_YEAR + 1]
    tci3 = target_channels_indices.reshape(B, 1, T)
    nsb = S // _SBLK
    out1, out2, out3 = pl.pallas_call(
        _expand_body,
        grid=(B, nsb),
        in_specs=[
            pl.BlockSpec((1, _SBLK, C), lambda b, s: (b, s, 0)),
            pl.BlockSpec((1, 1, C), lambda b, s: (b, 0, 0)),
            pl.BlockSpec((1, 1, C), lambda b, s: (b, 0, 0)),
            pl.BlockSpec((1, _SBLK, 4), lambda b, s: (b, s, 0)),
            pl.BlockSpec((1, P, 4), lambda b, s: (b, 0, 0)),
            pl.BlockSpec((1, P, T), lambda b, s: (b, 0, 0)),
            pl.BlockSpec((1, 1, T), lambda b, s: (b, 0, 0)),
            pl.BlockSpec((1, 1, 1), lambda b, s: (b, 0, 0)),
            pl.BlockSpec((_F, C * _E), lambda b, s: (0, 0)),
            pl.BlockSpec((11, 4), lambda b, s: (0, 0)),
            pl.BlockSpec((12, 4), lambda b, s: (0, 0)),
            pl.BlockSpec((31, 6), lambda b, s: (0, 0)),
            pl.BlockSpec((7, 4), lambda b, s: (0, 0)),
        ],
        out_specs=[
            pl.BlockSpec((1, _SBLK, C * _E), lambda b, s: (b, s, 0)),
            pl.BlockSpec((1, P, C * _E), lambda b, s: (b, 0, 0)),
            pl.BlockSpec((1, P, T), lambda b, s: (b, 0, 0)),
        ],
        out_shape=[
            jax.ShapeDtypeStruct((B, S, C * _E), jnp.float32),
            jax.ShapeDtypeStruct((B, P, C * _E), jnp.float32),
            jax.ShapeDtypeStruct((B, P, T), jnp.float32),
        ],
    )(history_values, med_bc, iqr_bc, history_time_features,
      target_time_features, target_values, tci3, ref_year, m,
      pos_year_table, pos_month_table, pos_day_table, pos_dow_table)

    return out1, out2.reshape(B, P, C, _E), out3


# speedup vs baseline: 3.1031x; 1.0007x over previous
import jax
import jax.numpy as jnp
from jax.experimental import pallas as pl

_YEAR, _MONTH, _DAY, _DOW = 0, 1, 2, 3
_EPS = 0.001
_E = 18
_C = 32
_F = 64

_SBLK = 1024
_LBLK = 128


def _chunk_stage(y, d, k=None, desc=None):
    n, lanes = y.shape
    g = n // (2 * d)
    xr = y.reshape(g, 2, d, lanes)
    a = xr[:, 0]
    b = xr[:, 1]
    lo = jnp.minimum(a, b)
    hi = jnp.maximum(a, b)
    if desc is None:
        gi = jax.lax.broadcasted_iota(jnp.int32, (g, 1, 1), 0)
        dm = ((gi >> (k - 1 - (d.bit_length() - 1))) & 1) == 1
        na = jnp.where(dm, hi, lo)
        nb = jnp.where(dm, lo, hi)
    elif desc:
        na, nb = hi, lo
    else:
        na, nb = lo, hi
    return jnp.stack([na, nb], axis=1).reshape(n, lanes)


def _sort128(y, final_desc):
    for k in range(1, 8):
        for j in range(k - 1, -1, -1):
            d = 1 << j
            if k == 7:
                y = _chunk_stage(y, d, desc=final_desc)
            else:
                y = _chunk_stage(y, d, k=k)
    return y


def _merge128(y, desc):
    for j in range(6, -1, -1):
        y = _chunk_stage(y, 1 << j, desc=desc)
    return y


def _minmax(a, b, desc):
    lo = jnp.minimum(a, b)
    hi = jnp.maximum(a, b)
    return (hi, lo) if desc else (lo, hi)


def _stats_body(hv_ref, out_ref):
    xb = hv_ref[...]
    x = jnp.concatenate([xb[0], xb[1], xb[2], xb[3]], axis=-1)
    lanes = x.shape[1]
    ch = [x[c * 128:(c + 1) * 128, :] for c in range(8)]
    ch = [_sort128(ch[c], final_desc=bool(c & 1)) for c in range(8)]
    for p in range(4):
        desc = bool(p & 1)
        a, b = _minmax(ch[2 * p], ch[2 * p + 1], desc)
        ch[2 * p] = _merge128(a, desc)
        ch[2 * p + 1] = _merge128(b, desc)
    for g2 in range(2):
        desc = bool(g2 & 1)
        base = g2 * 4
        for t in range(2):
            ch[base + t], ch[base + t + 2] = _minmax(ch[base + t], ch[base + t + 2], desc)
        for t in (0, 2):
            ch[base + t], ch[base + t + 1] = _minmax(ch[base + t], ch[base + t + 1], desc)
        for t in range(4):
            ch[base + t] = _merge128(ch[base + t], desc)
    for t in range(4):
        ch[t], ch[t + 4] = _minmax(ch[t], ch[t + 4], False)
    for t in (0, 1, 4, 5):
        ch[t], ch[t + 2] = _minmax(ch[t], ch[t + 2], False)
    for t in (0, 2, 4, 6):
        ch[t], ch[t + 1] = _minmax(ch[t], ch[t + 1], False)
    ch = [_merge128(c, False) for c in ch]
    med = 0.5 * (ch[3][127:128, :] + ch[4][0:1, :])
    q1 = 0.25 * ch[1][127:128, :] + 0.75 * ch[2][0:1, :]
    q3 = 0.75 * ch[5][127:128, :] + 0.25 * ch[6][0:1, :]
    iqr = (q3 - q1) + _EPS
    for i in range(lanes // _C):
        sl = slice(i * _C, (i + 1) * _C)
        out_ref[i, 0:1, :] = med[:, sl]
        out_ref[i, 1:2, :] = iqr[:, sl]
        out_ref[i, 2:8, :] = jnp.broadcast_to(med[:, sl], (6, _C))


_NOH = 7


def _onehot_feats(tf, ref_year):
    rows = tf.shape[0]
    dy = jnp.clip(ref_year - tf[:, _YEAR:_YEAR + 1], 0, 10)
    vals = jnp.concatenate(
        [jnp.broadcast_to(v, (rows, _NOH)) for v in
         (dy, tf[:, _MONTH:_MONTH + 1], tf[:, _DAY:_DAY + 1],
          tf[:, _DOW:_DOW + 1])], axis=1)
    slot = jax.lax.broadcasted_iota(jnp.int32, (rows, 4 * _NOH), 1) % _NOH
    return jnp.where(vals == slot, 1.0, 0.0)


def _pos_embed_block(tf, ref_year, yt_ref, mt_ref, dt_ref, wt_ref):
    year = tf[:, _YEAR:_YEAR + 1]
    month = tf[:, _MONTH:_MONTH + 1]
    day = tf[:, _DAY:_DAY + 1]
    dow = tf[:, _DOW:_DOW + 1]
    dy = jnp.clip(ref_year - year, 0, 10)

    def lookup(idx, table_ref, rows, width):
        acc = jnp.zeros((idx.shape[0], width), jnp.float32)
        for v in range(rows):
            row = table_ref[v:v + 1, :]
            acc = acc + jnp.where(idx == v, 1.0, 0.0) * row
        return acc

    pe_y = lookup(dy, yt_ref, 11, 4)
    pe_m = lookup(month, mt_ref, 12, 4)
    pe_d = lookup(day, dt_ref, 31, 6)
    pe_w = lookup(dow, wt_ref, 7, 4)
    return pe_y, pe_m, pe_d, pe_w


def _expand_body(hv_ref, st_ref, tf_ref, ttf_ref, tv_ref, tci_ref,
                 ry_ref, m_ref, yt_ref, mt_ref, dt_ref, wt_ref,
                 out1_ref, out2_ref, out3_ref):
    s = pl.program_id(1)
    ref_year = ry_ref[0, 0, 0]
    med = st_ref[0, 0:1, :]
    iqr = st_ref[0, 1:2, :]
    m = m_ref[...]

    hv = hv_ref[0]
    hs = (hv - med) / iqr
    rows = hs.shape[0]
    oh = _onehot_feats(tf_ref[0], ref_year)
    feats = jnp.concatenate(
        [hs, oh, jnp.ones((rows, 1), jnp.float32),
         jnp.zeros((rows, _F - _C - 4 * _NOH - 1), jnp.float32)], axis=-1)
    out1_ref[0] = jax.lax.dot(
        feats, m, precision=jax.lax.Precision.HIGHEST,
        preferred_element_type=jnp.float32)

    @pl.when(s == 0)
    def _targets():
        toh = _onehot_feats(ttf_ref[0], ref_year)
        p = toh.shape[0]
        tfeats = jnp.concatenate(
            [jnp.zeros((p, _C), jnp.float32), toh,
             jnp.zeros((p, _F - _C - 4 * _NOH), jnp.float32)], axis=-1)
        out2_ref[0] = jax.lax.dot(
            tfeats, m, precision=jax.lax.Precision.HIGHEST,
            preferred_element_type=jnp.float32)
        tci = tci_ref[0]
        med_g = jnp.zeros(tci.shape, jnp.float32)
        iqr_g = jnp.zeros(tci.shape, jnp.float32)
        for c in range(_C):
            hit = jnp.where(tci == c, 1.0, 0.0)
            med_g = med_g + hit * med[:, c:c + 1]
            iqr_g = iqr_g + hit * iqr[:, c:c + 1]
        out3_ref[0] = (tv_ref[0] - med_g) / iqr_g


def _build_m(w_expand, b_expand, yt, mt, dt, wt):
    ce = jnp.arange(_C * _E)
    cidx = ce // _E
    rows32 = jnp.arange(_C)[:, None]
    wfull = jnp.tile(w_expand, _C)[None, :]
    m_w = jnp.where(rows32 == cidx[None, :], wfull, 0.0)
    blocks = [m_w]
    for tab, off in ((yt, 0), (mt, 4), (dt, 8), (wt, 14)):
        width = tab.shape[1]
        padded = jnp.pad(tab[:_NOH], ((0, 0), (off, _E - off - width)))
        blocks.append(jnp.tile(padded, (1, _C)))
    blocks.append(jnp.tile(b_expand, _C)[None, :])
    blocks.append(jnp.zeros((_F - _C - 4 * _NOH - 1, _C * _E)))
    return jnp.concatenate(blocks, axis=0).astype(jnp.float32)


@jax.jit
def kernel(history_values, target_values, target_channels_indices,
           history_time_features, target_time_features,
           pos_year_table, pos_month_table, pos_day_table, pos_dow_table,
           W_expand, b_expand):
    B, S, C = history_values.shape
    P, T = target_values.shape[1], target_values.shape[2]

    nlb = (B * C) // _LBLK
    stats = pl.pallas_call(
        _stats_body,
        grid=(nlb,),
        in_specs=[pl.BlockSpec((_LBLK // C, S, C), lambda i: (i, 0, 0))],
        out_specs=pl.BlockSpec((_LBLK // C, 8, C), lambda i: (i, 0, 0)),
        out_shape=jax.ShapeDtypeStruct((B, 8, C), jnp.float32),
    )(history_values)

    m = _build_m(W_expand, b_expand, pos_year_table, pos_month_table,
                 pos_day_table, pos_dow_table)
    ref_year = history_time_features[:, S - 1:S, _YEAR:_YEAR + 1]
    tci3 = target_channels_indices.reshape(B, 1, T)
    nsb = S // _SBLK
    out1, out2, out3 = pl.pallas_call(
        _expand_body,
        grid=(B, nsb),
        in_specs=[
            pl.BlockSpec((1, _SBLK, C), lambda b, s: (b, s, 0)),
            pl.BlockSpec((1, 8, C), lambda b, s: (b, 0, 0)),
            pl.BlockSpec((1, _SBLK, 4), lambda b, s: (b, s, 0)),
            pl.BlockSpec((1, P, 4), lambda b, s: (b, 0, 0)),
            pl.BlockSpec((1, P, T), lambda b, s: (b, 0, 0)),
            pl.BlockSpec((1, 1, T), lambda b, s: (b, 0, 0)),
            pl.BlockSpec((1, 1, 1), lambda b, s: (b, 0, 0)),
            pl.BlockSpec((_F, C * _E), lambda b, s: (0, 0)),
            pl.BlockSpec((11, 4), lambda b, s: (0, 0)),
            pl.BlockSpec((12, 4), lambda b, s: (0, 0)),
            pl.BlockSpec((31, 6), lambda b, s: (0, 0)),
            pl.BlockSpec((7, 4), lambda b, s: (0, 0)),
        ],
        out_specs=[
            pl.BlockSpec((1, _SBLK, C * _E), lambda b, s: (b, s, 0)),
            pl.BlockSpec((1, P, C * _E), lambda b, s: (b, 0, 0)),
            pl.BlockSpec((1, P, T), lambda b, s: (b, 0, 0)),
        ],
        out_shape=[
            jax.ShapeDtypeStruct((B, S, C * _E), jnp.float32),
            jax.ShapeDtypeStruct((B, P, C * _E), jnp.float32),
            jax.ShapeDtypeStruct((B, P, T), jnp.float32),
        ],
    )(history_values, stats, history_time_features,
      target_time_features, target_values, tci3, ref_year, m,
      pos_year_table, pos_month_table, pos_day_table, pos_dow_table)

    return out1, out2.reshape(B, P, C, _E), out3


# speedup vs baseline: 3.6411x; 1.1734x over previous
import jax
import jax.numpy as jnp
from jax.experimental import pallas as pl

_YEAR, _MONTH, _DAY, _DOW = 0, 1, 2, 3
_EPS = 0.001
_E = 18
_C = 32
_F = 64

_SBLK = 1024
_LBLK = 128


def _chunk_stage(y, d, k=None, desc=None):
    n, lanes = y.shape
    g = n // (2 * d)
    xr = y.reshape(g, 2, d, lanes)
    a = xr[:, 0]
    b = xr[:, 1]
    lo = jnp.minimum(a, b)
    hi = jnp.maximum(a, b)
    if desc is None:
        gi = jax.lax.broadcasted_iota(jnp.int32, (g, 1, 1), 0)
        dm = ((gi >> (k - 1 - (d.bit_length() - 1))) & 1) == 1
        na = jnp.where(dm, hi, lo)
        nb = jnp.where(dm, lo, hi)
    elif desc:
        na, nb = hi, lo
    else:
        na, nb = lo, hi
    return jnp.stack([na, nb], axis=1).reshape(n, lanes)


def _sort128(y, final_desc):
    for k in range(1, 8):
        for j in range(k - 1, -1, -1):
            d = 1 << j
            if k == 7:
                y = _chunk_stage(y, d, desc=final_desc)
            else:
                y = _chunk_stage(y, d, k=k)
    return y


def _merge128(y, desc):
    for j in range(6, -1, -1):
        y = _chunk_stage(y, 1 << j, desc=desc)
    return y


def _minmax(a, b, desc):
    lo = jnp.minimum(a, b)
    hi = jnp.maximum(a, b)
    return (hi, lo) if desc else (lo, hi)


def _stats_body(hv_ref, out_ref):
    xb = hv_ref[...]
    x = jnp.concatenate([xb[0], xb[1], xb[2], xb[3]], axis=-1)
    lanes = x.shape[1]
    ch = [x[c * 128:(c + 1) * 128, :] for c in range(8)]
    ch = [_sort128(ch[c], final_desc=bool(c & 1)) for c in range(8)]
    for p in range(4):
        desc = bool(p & 1)
        a, b = _minmax(ch[2 * p], ch[2 * p + 1], desc)
        ch[2 * p] = _merge128(a, desc)
        ch[2 * p + 1] = _merge128(b, desc)
    for g2 in range(2):
        desc = bool(g2 & 1)
        base = g2 * 4
        for t in range(2):
            ch[base + t], ch[base + t + 2] = _minmax(ch[base + t], ch[base + t + 2], desc)
        for t in (0, 2):
            ch[base + t], ch[base + t + 1] = _minmax(ch[base + t], ch[base + t + 1], desc)
        for t in range(4):
            ch[base + t] = _merge128(ch[base + t], desc)
    for t in range(4):
        ch[t], ch[t + 4] = _minmax(ch[t], ch[t + 4], False)
    for t in (0, 1, 4, 5):
        ch[t], ch[t + 2] = _minmax(ch[t], ch[t + 2], False)
    for t in (0, 2, 4, 6):
        ch[t], ch[t + 1] = _minmax(ch[t], ch[t + 1], False)
    ch = [_merge128(c, False) for c in ch]
    med = 0.5 * (ch[3][127:128, :] + ch[4][0:1, :])
    q1 = 0.25 * ch[1][127:128, :] + 0.75 * ch[2][0:1, :]
    q3 = 0.75 * ch[5][127:128, :] + 0.25 * ch[6][0:1, :]
    iqr = (q3 - q1) + _EPS
    for i in range(lanes // _C):
        sl = slice(i * _C, (i + 1) * _C)
        out_ref[i, 0:1, :] = med[:, sl]
        out_ref[i, 1:2, :] = iqr[:, sl]
        out_ref[i, 2:8, :] = jnp.broadcast_to(med[:, sl], (6, _C))


_NOH = 7


def _onehot_feats(tf, ref_year):
    rows = tf.shape[0]
    dy = jnp.clip(ref_year - tf[:, _YEAR:_YEAR + 1], 0, 10)
    vals = jnp.concatenate(
        [jnp.broadcast_to(v, (rows, _NOH)) for v in
         (dy, tf[:, _MONTH:_MONTH + 1], tf[:, _DAY:_DAY + 1],
          tf[:, _DOW:_DOW + 1])], axis=1)
    slot = jax.lax.broadcasted_iota(jnp.int32, (rows, 4 * _NOH), 1) % _NOH
    return jnp.where(vals == slot, 1.0, 0.0)


def _pos_embed_block(tf, ref_year, yt_ref, mt_ref, dt_ref, wt_ref):
    year = tf[:, _YEAR:_YEAR + 1]
    month = tf[:, _MONTH:_MONTH + 1]
    day = tf[:, _DAY:_DAY + 1]
    dow = tf[:, _DOW:_DOW + 1]
    dy = jnp.clip(ref_year - year, 0, 10)

    def lookup(idx, table_ref, rows, width):
        acc = jnp.zeros((idx.shape[0], width), jnp.float32)
        for v in range(rows):
            row = table_ref[v:v + 1, :]
            acc = acc + jnp.where(idx == v, 1.0, 0.0) * row
        return acc

    pe_y = lookup(dy, yt_ref, 11, 4)
    pe_m = lookup(month, mt_ref, 12, 4)
    pe_d = lookup(day, dt_ref, 31, 6)
    pe_w = lookup(dow, wt_ref, 7, 4)
    return pe_y, pe_m, pe_d, pe_w


def _expand_body(hv_ref, st_ref, tf_ref, ttf_ref, tv_ref, tci_ref,
                 ry_ref, m_ref, yt_ref, mt_ref, dt_ref, wt_ref,
                 out1_ref, out2_ref, out3_ref):
    s = pl.program_id(1)
    ref_year = ry_ref[0, 0, 0]
    med = st_ref[0, 0:1, :]
    iqr = st_ref[0, 1:2, :]
    m = m_ref[...]

    hv = hv_ref[0]
    hs = (hv - med) / iqr
    rows = hs.shape[0]
    oh = _onehot_feats(tf_ref[0], ref_year)
    feats = jnp.concatenate(
        [hs, oh, jnp.ones((rows, 1), jnp.float32),
         jnp.zeros((rows, _F - _C - 4 * _NOH - 1), jnp.float32)], axis=-1)
    out1_ref[0] = jax.lax.dot(
        feats, m, precision=jax.lax.Precision.DEFAULT,
        preferred_element_type=jnp.float32)

    @pl.when(s == 0)
    def _targets():
        toh = _onehot_feats(ttf_ref[0], ref_year)
        p = toh.shape[0]
        tfeats = jnp.concatenate(
            [jnp.zeros((p, _C), jnp.float32), toh,
             jnp.zeros((p, _F - _C - 4 * _NOH), jnp.float32)], axis=-1)
        out2_ref[0] = jax.lax.dot(
            tfeats, m, precision=jax.lax.Precision.DEFAULT,
            preferred_element_type=jnp.float32)
        tci = tci_ref[0]
        med_g = jnp.zeros(tci.shape, jnp.float32)
        iqr_g = jnp.zeros(tci.shape, jnp.float32)
        for c in range(_C):
            hit = jnp.where(tci == c, 1.0, 0.0)
            med_g = med_g + hit * med[:, c:c + 1]
            iqr_g = iqr_g + hit * iqr[:, c:c + 1]
        out3_ref[0] = (tv_ref[0] - med_g) / iqr_g


def _build_m(w_expand, b_expand, yt, mt, dt, wt):
    ce = jnp.arange(_C * _E)
    cidx = ce // _E
    rows32 = jnp.arange(_C)[:, None]
    wfull = jnp.tile(w_expand, _C)[None, :]
    m_w = jnp.where(rows32 == cidx[None, :], wfull, 0.0)
    blocks = [m_w]
    for tab, off in ((yt, 0), (mt, 4), (dt, 8), (wt, 14)):
        width = tab.shape[1]
        padded = jnp.pad(tab[:_NOH], ((0, 0), (off, _E - off - width)))
        blocks.append(jnp.tile(padded, (1, _C)))
    blocks.append(jnp.tile(b_expand, _C)[None, :])
    blocks.append(jnp.zeros((_F - _C - 4 * _NOH - 1, _C * _E)))
    return jnp.concatenate(blocks, axis=0).astype(jnp.float32)


@jax.jit
def kernel(history_values, target_values, target_channels_indices,
           history_time_features, target_time_features,
           pos_year_table, pos_month_table, pos_day_table, pos_dow_table,
           W_expand, b_expand):
    B, S, C = history_values.shape
    P, T = target_values.shape[1], target_values.shape[2]

    nlb = (B * C) // _LBLK
    stats = pl.pallas_call(
        _stats_body,
        grid=(nlb,),
        in_specs=[pl.BlockSpec((_LBLK // C, S, C), lambda i: (i, 0, 0))],
        out_specs=pl.BlockSpec((_LBLK // C, 8, C), lambda i: (i, 0, 0)),
        out_shape=jax.ShapeDtypeStruct((B, 8, C), jnp.float32),
    )(history_values)

    m = _build_m(W_expand, b_expand, pos_year_table, pos_month_table,
                 pos_day_table, pos_dow_table)
    ref_year = history_time_features[:, S - 1:S, _YEAR:_YEAR + 1]
    tci3 = target_channels_indices.reshape(B, 1, T)
    nsb = S // _SBLK
    out1, out2, out3 = pl.pallas_call(
        _expand_body,
        grid=(B, nsb),
        in_specs=[
            pl.BlockSpec((1, _SBLK, C), lambda b, s: (b, s, 0)),
            pl.BlockSpec((1, 8, C), lambda b, s: (b, 0, 0)),
            pl.BlockSpec((1, _SBLK, 4), lambda b, s: (b, s, 0)),
            pl.BlockSpec((1, P, 4), lambda b, s: (b, 0, 0)),
            pl.BlockSpec((1, P, T), lambda b, s: (b, 0, 0)),
            pl.BlockSpec((1, 1, T), lambda b, s: (b, 0, 0)),
            pl.BlockSpec((1, 1, 1), lambda b, s: (b, 0, 0)),
            pl.BlockSpec((_F, C * _E), lambda b, s: (0, 0)),
            pl.BlockSpec((11, 4), lambda b, s: (0, 0)),
            pl.BlockSpec((12, 4), lambda b, s: (0, 0)),
            pl.BlockSpec((31, 6), lambda b, s: (0, 0)),
            pl.BlockSpec((7, 4), lambda b, s: (0, 0)),
        ],
        out_specs=[
            pl.BlockSpec((1, _SBLK, C * _E), lambda b, s: (b, s, 0)),
            pl.BlockSpec((1, P, C * _E), lambda b, s: (b, 0, 0)),
            pl.BlockSpec((1, P, T), lambda b, s: (b, 0, 0)),
        ],
        out_shape=[
            jax.ShapeDtypeStruct((B, S, C * _E), jnp.float32),
            jax.ShapeDtypeStruct((B, P, C * _E), jnp.float32),
            jax.ShapeDtypeStruct((B, P, T), jnp.float32),
        ],
    )(history_values, stats, history_time_features,
      target_time_features, target_values, tci3, ref_year, m,
      pos_year_table, pos_month_table, pos_day_table, pos_dow_table)

    return out1, out2.reshape(B, P, C, _E), out3


# speedup vs baseline: 4.5231x; 1.2422x over previous
import jax
import jax.numpy as jnp
from jax.experimental import pallas as pl

_YEAR, _MONTH, _DAY, _DOW = 0, 1, 2, 3
_EPS = 0.001
_E = 18
_C = 32
_F = 64

_SBLK = 1024
_LBLK = 256


def _chunk_stage(y, d, k=None, desc=None):
    n, lanes = y.shape
    g = n // (2 * d)
    xr = y.reshape(g, 2, d, lanes)
    a = xr[:, 0]
    b = xr[:, 1]
    lo = jnp.minimum(a, b)
    hi = jnp.maximum(a, b)
    if desc is None:
        gi = jax.lax.broadcasted_iota(jnp.int32, (g, 1, 1), 0)
        dm = ((gi >> (k - 1 - (d.bit_length() - 1))) & 1) == 1
        na = jnp.where(dm, hi, lo)
        nb = jnp.where(dm, lo, hi)
    elif desc:
        na, nb = hi, lo
    else:
        na, nb = lo, hi
    return jnp.stack([na, nb], axis=1).reshape(n, lanes)


def _sort128(y, final_desc):
    for k in range(1, 8):
        for j in range(k - 1, -1, -1):
            d = 1 << j
            if k == 7:
                y = _chunk_stage(y, d, desc=final_desc)
            else:
                y = _chunk_stage(y, d, k=k)
    return y


def _merge128(y, desc):
    for j in range(6, -1, -1):
        y = _chunk_stage(y, 1 << j, desc=desc)
    return y


def _minmax(a, b, desc):
    lo = jnp.minimum(a, b)
    hi = jnp.maximum(a, b)
    return (hi, lo) if desc else (lo, hi)


def _stats_body(hv_ref, out_ref):
    xb = hv_ref[...]
    x = jnp.concatenate([xb[0], xb[1], xb[2], xb[3]], axis=-1)
    lanes = x.shape[1]
    ch = [x[c * 128:(c + 1) * 128, :] for c in range(8)]
    ch = [_sort128(ch[c], final_desc=bool(c & 1)) for c in range(8)]
    for p in range(4):
        desc = bool(p & 1)
        a, b = _minmax(ch[2 * p], ch[2 * p + 1], desc)
        ch[2 * p] = _merge128(a, desc)
        ch[2 * p + 1] = _merge128(b, desc)
    for g2 in range(2):
        desc = bool(g2 & 1)
        base = g2 * 4
        for t in range(2):
            ch[base + t], ch[base + t + 2] = _minmax(ch[base + t], ch[base + t + 2], desc)
        for t in (0, 2):
            ch[base + t], ch[base + t + 1] = _minmax(ch[base + t], ch[base + t + 1], desc)
        for t in range(4):
            ch[base + t] = _merge128(ch[base + t], desc)
    for t in range(4):
        ch[t], ch[t + 4] = _minmax(ch[t], ch[t + 4], False)
    for t in (0, 1, 4, 5):
        ch[t], ch[t + 2] = _minmax(ch[t], ch[t + 2], False)
    for t in (0, 2, 4, 6):
        ch[t], ch[t + 1] = _minmax(ch[t], ch[t + 1], False)
    ch = [_merge128(c, False) for c in ch]
    med = 0.5 * (ch[3][127:128, :] + ch[4][0:1, :])
    q1 = 0.25 * ch[1][127:128, :] + 0.75 * ch[2][0:1, :]
    q3 = 0.75 * ch[5][127:128, :] + 0.25 * ch[6][0:1, :]
    iqr = (q3 - q1) + _EPS
    for i in range(lanes // _C):
        sl = slice(i * _C, (i + 1) * _C)
        out_ref[i, 0:1, :] = med[:, sl]
        out_ref[i, 1:2, :] = iqr[:, sl]
        out_ref[i, 2:8, :] = jnp.broadcast_to(med[:, sl], (6, _C))


_NOH = 7


def _onehot_feats(tf, ref_year):
    rows = tf.shape[0]
    dy = jnp.clip(ref_year - tf[:, _YEAR:_YEAR + 1], 0, 10)
    vals = jnp.concatenate(
        [jnp.broadcast_to(v, (rows, _NOH)) for v in
         (dy, tf[:, _MONTH:_MONTH + 1], tf[:, _DAY:_DAY + 1],
          tf[:, _DOW:_DOW + 1])], axis=1)
    slot = jax.lax.broadcasted_iota(jnp.int32, (rows, 4 * _NOH), 1) % _NOH
    return jnp.where(vals == slot, 1.0, 0.0)


def _pos_embed_block(tf, ref_year, yt_ref, mt_ref, dt_ref, wt_ref):
    year = tf[:, _YEAR:_YEAR + 1]
    month = tf[:, _MONTH:_MONTH + 1]
    day = tf[:, _DAY:_DAY + 1]
    dow = tf[:, _DOW:_DOW + 1]
    dy = jnp.clip(ref_year - year, 0, 10)

    def lookup(idx, table_ref, rows, width):
        acc = jnp.zeros((idx.shape[0], width), jnp.float32)
        for v in range(rows):
            row = table_ref[v:v + 1, :]
            acc = acc + jnp.where(idx == v, 1.0, 0.0) * row
        return acc

    pe_y = lookup(dy, yt_ref, 11, 4)
    pe_m = lookup(month, mt_ref, 12, 4)
    pe_d = lookup(day, dt_ref, 31, 6)
    pe_w = lookup(dow, wt_ref, 7, 4)
    return pe_y, pe_m, pe_d, pe_w


def _expand_body(hv_ref, st_ref, tf_ref, ttf_ref, tv_ref, tci_ref,
                 ry_ref, m_ref, yt_ref, mt_ref, dt_ref, wt_ref,
                 out1_ref, out2_ref, out3_ref):
    s = pl.program_id(1)
    ref_year = ry_ref[0, 0, 0]
    med = st_ref[0, 0:1, :]
    iqr = st_ref[0, 1:2, :]
    m = m_ref[...]

    hv = hv_ref[0]
    hs = (hv - med) / iqr
    rows = hs.shape[0]
    oh = _onehot_feats(tf_ref[0], ref_year)
    feats = jnp.concatenate(
        [hs, oh, jnp.ones((rows, 1), jnp.float32),
         jnp.zeros((rows, _F - _C - 4 * _NOH - 1), jnp.float32)], axis=-1)
    out1_ref[0] = jax.lax.dot(
        feats, m, precision=jax.lax.Precision.DEFAULT,
        preferred_element_type=jnp.float32)

    @pl.when(s == 0)
    def _targets():
        toh = _onehot_feats(ttf_ref[0], ref_year)
        p = toh.shape[0]
        tfeats = jnp.concatenate(
            [jnp.zeros((p, _C), jnp.float32), toh,
             jnp.zeros((p, _F - _C - 4 * _NOH), jnp.float32)], axis=-1)
        out2_ref[0] = jax.lax.dot(
            tfeats, m, precision=jax.lax.Precision.DEFAULT,
            preferred_element_type=jnp.float32)
        tci = tci_ref[0]
        med_g = jnp.zeros(tci.shape, jnp.float32)
        iqr_g = jnp.zeros(tci.shape, jnp.float32)
        for c in range(_C):
            hit = jnp.where(tci == c, 1.0, 0.0)
            med_g = med_g + hit * med[:, c:c + 1]
            iqr_g = iqr_g + hit * iqr[:, c:c + 1]
        out3_ref[0] = (tv_ref[0] - med_g) / iqr_g


def _build_m(w_expand, b_expand, yt, mt, dt, wt):
    ce = jnp.arange(_C * _E)
    cidx = ce // _E
    rows32 = jnp.arange(_C)[:, None]
    wfull = jnp.tile(w_expand, _C)[None, :]
    m_w = jnp.where(rows32 == cidx[None, :], wfull, 0.0)
    blocks = [m_w]
    for tab, off in ((yt, 0), (mt, 4), (dt, 8), (wt, 14)):
        width = tab.shape[1]
        padded = jnp.pad(tab[:_NOH], ((0, 0), (off, _E - off - width)))
        blocks.append(jnp.tile(padded, (1, _C)))
    blocks.append(jnp.tile(b_expand, _C)[None, :])
    blocks.append(jnp.zeros((_F - _C - 4 * _NOH - 1, _C * _E)))
    return jnp.concatenate(blocks, axis=0).astype(jnp.float32)


@jax.jit
def kernel(history_values, target_values, target_channels_indices,
           history_time_features, target_time_features,
           pos_year_table, pos_month_table, pos_day_table, pos_dow_table,
           W_expand, b_expand):
    B, S, C = history_values.shape
    P, T = target_values.shape[1], target_values.shape[2]

    nlb = (B * C) // _LBLK
    stats = pl.pallas_call(
        _stats_body,
        grid=(nlb,),
        in_specs=[pl.BlockSpec((_LBLK // C, S, C), lambda i: (i, 0, 0))],
        out_specs=pl.BlockSpec((_LBLK // C, 8, C), lambda i: (i, 0, 0)),
        out_shape=jax.ShapeDtypeStruct((B, 8, C), jnp.float32),
    )(history_values)

    m = _build_m(W_expand, b_expand, pos_year_table, pos_month_table,
                 pos_day_table, pos_dow_table)
    ref_year = history_time_features[:, S - 1:S, _YEAR:_YEAR + 1]
    tci3 = target_channels_indices.reshape(B, 1, T)
    nsb = S // _SBLK
    out1, out2, out3 = pl.pallas_call(
        _expand_body,
        grid=(B, nsb),
        in_specs=[
            pl.BlockSpec((1, _SBLK, C), lambda b, s: (b, s, 0)),
            pl.BlockSpec((1, 8, C), lambda b, s: (b, 0, 0)),
            pl.BlockSpec((1, _SBLK, 4), lambda b, s: (b, s, 0)),
            pl.BlockSpec((1, P, 4), lambda b, s: (b, 0, 0)),
            pl.BlockSpec((1, P, T), lambda b, s: (b, 0, 0)),
            pl.BlockSpec((1, 1, T), lambda b, s: (b, 0, 0)),
            pl.BlockSpec((1, 1, 1), lambda b, s: (b, 0, 0)),
            pl.BlockSpec((_F, C * _E), lambda b, s: (0, 0)),
            pl.BlockSpec((11, 4), lambda b, s: (0, 0)),
            pl.BlockSpec((12, 4), lambda b, s: (0, 0)),
            pl.BlockSpec((31, 6), lambda b, s: (0, 0)),
            pl.BlockSpec((7, 4), lambda b, s: (0, 0)),
        ],
        out_specs=[
            pl.BlockSpec((1, _SBLK, C * _E), lambda b, s: (b, s, 0)),
            pl.BlockSpec((1, P, C * _E), lambda b, s: (b, 0, 0)),
            pl.BlockSpec((1, P, T), lambda b, s: (b, 0, 0)),
        ],
        out_shape=[
            jax.ShapeDtypeStruct((B, S, C * _E), jnp.float32),
            jax.ShapeDtypeStruct((B, P, C * _E), jnp.float32),
            jax.ShapeDtypeStruct((B, P, T), jnp.float32),
        ],
    )(history_values, stats, history_time_features,
      target_time_features, target_values, tci3, ref_year, m,
      pos_year_table, pos_month_table, pos_day_table, pos_dow_table)

    return out1, out2.reshape(B, P, C, _E), out3


# speedup vs baseline: 5.1409x; 1.1366x over previous
import jax
import jax.numpy as jnp
from jax.experimental import pallas as pl

_YEAR, _MONTH, _DAY, _DOW = 0, 1, 2, 3
_EPS = 0.001
_E = 18
_C = 32
_F = 64

_SBLK = 1024
_LBLK = 512


def _chunk_stage(y, d, k=None, desc=None):
    n, lanes = y.shape
    g = n // (2 * d)
    xr = y.reshape(g, 2, d, lanes)
    a = xr[:, 0]
    b = xr[:, 1]
    lo = jnp.minimum(a, b)
    hi = jnp.maximum(a, b)
    if desc is None:
        gi = jax.lax.broadcasted_iota(jnp.int32, (g, 1, 1), 0)
        dm = ((gi >> (k - 1 - (d.bit_length() - 1))) & 1) == 1
        na = jnp.where(dm, hi, lo)
        nb = jnp.where(dm, lo, hi)
    elif desc:
        na, nb = hi, lo
    else:
        na, nb = lo, hi
    return jnp.stack([na, nb], axis=1).reshape(n, lanes)


def _sort128(y, final_desc):
    for k in range(1, 8):
        for j in range(k - 1, -1, -1):
            d = 1 << j
            if k == 7:
                y = _chunk_stage(y, d, desc=final_desc)
            else:
                y = _chunk_stage(y, d, k=k)
    return y


def _merge128(y, desc):
    for j in range(6, -1, -1):
        y = _chunk_stage(y, 1 << j, desc=desc)
    return y


def _minmax(a, b, desc):
    lo = jnp.minimum(a, b)
    hi = jnp.maximum(a, b)
    return (hi, lo) if desc else (lo, hi)


def _stats_body(hv_ref, out_ref):
    xb = hv_ref[...]
    x = jnp.concatenate([xb[0], xb[1], xb[2], xb[3]], axis=-1)
    lanes = x.shape[1]
    ch = [x[c * 128:(c + 1) * 128, :] for c in range(8)]
    ch = [_sort128(ch[c], final_desc=bool(c & 1)) for c in range(8)]
    for p in range(4):
        desc = bool(p & 1)
        a, b = _minmax(ch[2 * p], ch[2 * p + 1], desc)
        ch[2 * p] = _merge128(a, desc)
        ch[2 * p + 1] = _merge128(b, desc)
    for g2 in range(2):
        desc = bool(g2 & 1)
        base = g2 * 4
        for t in range(2):
            ch[base + t], ch[base + t + 2] = _minmax(ch[base + t], ch[base + t + 2], desc)
        for t in (0, 2):
            ch[base + t], ch[base + t + 1] = _minmax(ch[base + t], ch[base + t + 1], desc)
        for t in range(4):
            ch[base + t] = _merge128(ch[base + t], desc)
    for t in range(4):
        ch[t], ch[t + 4] = _minmax(ch[t], ch[t + 4], False)
    for t in (0, 1, 4, 5):
        ch[t], ch[t + 2] = _minmax(ch[t], ch[t + 2], False)
    for t in (0, 2, 4, 6):
        ch[t], ch[t + 1] = _minmax(ch[t], ch[t + 1], False)
    ch = [_merge128(c, False) for c in ch]
    med = 0.5 * (ch[3][127:128, :] + ch[4][0:1, :])
    q1 = 0.25 * ch[1][127:128, :] + 0.75 * ch[2][0:1, :]
    q3 = 0.75 * ch[5][127:128, :] + 0.25 * ch[6][0:1, :]
    iqr = (q3 - q1) + _EPS
    for i in range(lanes // _C):
        sl = slice(i * _C, (i + 1) * _C)
        out_ref[i, 0:1, :] = med[:, sl]
        out_ref[i, 1:2, :] = iqr[:, sl]
        out_ref[i, 2:8, :] = jnp.broadcast_to(med[:, sl], (6, _C))


_NOH = 7


def _onehot_feats(tf, ref_year):
    rows = tf.shape[0]
    dy = jnp.clip(ref_year - tf[:, _YEAR:_YEAR + 1], 0, 10)
    vals = jnp.concatenate(
        [jnp.broadcast_to(v, (rows, _NOH)) for v in
         (dy, tf[:, _MONTH:_MONTH + 1], tf[:, _DAY:_DAY + 1],
          tf[:, _DOW:_DOW + 1])], axis=1)
    slot = jax.lax.broadcasted_iota(jnp.int32, (rows, 4 * _NOH), 1) % _NOH
    return jnp.where(vals == slot, 1.0, 0.0)


def _pos_embed_block(tf, ref_year, yt_ref, mt_ref, dt_ref, wt_ref):
    year = tf[:, _YEAR:_YEAR + 1]
    month = tf[:, _MONTH:_MONTH + 1]
    day = tf[:, _DAY:_DAY + 1]
    dow = tf[:, _DOW:_DOW + 1]
    dy = jnp.clip(ref_year - year, 0, 10)

    def lookup(idx, table_ref, rows, width):
        acc = jnp.zeros((idx.shape[0], width), jnp.float32)
        for v in range(rows):
            row = table_ref[v:v + 1, :]
            acc = acc + jnp.where(idx == v, 1.0, 0.0) * row
        return acc

    pe_y = lookup(dy, yt_ref, 11, 4)
    pe_m = lookup(month, mt_ref, 12, 4)
    pe_d = lookup(day, dt_ref, 31, 6)
    pe_w = lookup(dow, wt_ref, 7, 4)
    return pe_y, pe_m, pe_d, pe_w


def _expand_body(hv_ref, st_ref, tf_ref, ttf_ref, tv_ref, tci_ref,
                 ry_ref, m_ref, yt_ref, mt_ref, dt_ref, wt_ref,
                 out1_ref, out2_ref, out3_ref):
    s = pl.program_id(1)
    ref_year = ry_ref[0, 0, 0]
    med = st_ref[0, 0:1, :]
    iqr = st_ref[0, 1:2, :]
    m = m_ref[...]

    hv = hv_ref[0]
    hs = (hv - med) / iqr
    rows = hs.shape[0]
    oh = _onehot_feats(tf_ref[0], ref_year)
    feats = jnp.concatenate(
        [hs, oh, jnp.ones((rows, 1), jnp.float32),
         jnp.zeros((rows, _F - _C - 4 * _NOH - 1), jnp.float32)], axis=-1)
    out1_ref[0] = jax.lax.dot(
        feats, m, precision=jax.lax.Precision.DEFAULT,
        preferred_element_type=jnp.float32)

    @pl.when(s == 0)
    def _targets():
        toh = _onehot_feats(ttf_ref[0], ref_year)
        p = toh.shape[0]
        tfeats = jnp.concatenate(
            [jnp.zeros((p, _C), jnp.float32), toh,
             jnp.zeros((p, _F - _C - 4 * _NOH), jnp.float32)], axis=-1)
        out2_ref[0] = jax.lax.dot(
            tfeats, m, precision=jax.lax.Precision.DEFAULT,
            preferred_element_type=jnp.float32)
        tci = tci_ref[0]
        med_g = jnp.zeros(tci.shape, jnp.float32)
        iqr_g = jnp.zeros(tci.shape, jnp.float32)
        for c in range(_C):
            hit = jnp.where(tci == c, 1.0, 0.0)
            med_g = med_g + hit * med[:, c:c + 1]
            iqr_g = iqr_g + hit * iqr[:, c:c + 1]
        out3_ref[0] = (tv_ref[0] - med_g) / iqr_g


def _build_m(w_expand, b_expand, yt, mt, dt, wt):
    ce = jnp.arange(_C * _E)
    cidx = ce // _E
    rows32 = jnp.arange(_C)[:, None]
    wfull = jnp.tile(w_expand, _C)[None, :]
    m_w = jnp.where(rows32 == cidx[None, :], wfull, 0.0)
    blocks = [m_w]
    for tab, off in ((yt, 0), (mt, 4), (dt, 8), (wt, 14)):
        width = tab.shape[1]
        padded = jnp.pad(tab[:_NOH], ((0, 0), (off, _E - off - width)))
        blocks.append(jnp.tile(padded, (1, _C)))
    blocks.append(jnp.tile(b_expand, _C)[None, :])
    blocks.append(jnp.zeros((_F - _C - 4 * _NOH - 1, _C * _E)))
    return jnp.concatenate(blocks, axis=0).astype(jnp.float32)


@jax.jit
def kernel(history_values, target_values, target_channels_indices,
           history_time_features, target_time_features,
           pos_year_table, pos_month_table, pos_day_table, pos_dow_table,
           W_expand, b_expand):
    B, S, C = history_values.shape
    P, T = target_values.shape[1], target_values.shape[2]

    nlb = (B * C) // _LBLK
    stats = pl.pallas_call(
        _stats_body,
        grid=(nlb,),
        in_specs=[pl.BlockSpec((_LBLK // C, S, C), lambda i: (i, 0, 0))],
        out_specs=pl.BlockSpec((_LBLK // C, 8, C), lambda i: (i, 0, 0)),
        out_shape=jax.ShapeDtypeStruct((B, 8, C), jnp.float32),
    )(history_values)

    m = _build_m(W_expand, b_expand, pos_year_table, pos_month_table,
                 pos_day_table, pos_dow_table)
    ref_year = history_time_features[:, S - 1:S, _YEAR:_YEAR + 1]
    tci3 = target_channels_indices.reshape(B, 1, T)
    nsb = S // _SBLK
    out1, out2, out3 = pl.pallas_call(
        _expand_body,
        grid=(B, nsb),
        in_specs=[
            pl.BlockSpec((1, _SBLK, C), lambda b, s: (b, s, 0)),
            pl.BlockSpec((1, 8, C), lambda b, s: (b, 0, 0)),
            pl.BlockSpec((1, _SBLK, 4), lambda b, s: (b, s, 0)),
            pl.BlockSpec((1, P, 4), lambda b, s: (b, 0, 0)),
            pl.BlockSpec((1, P, T), lambda b, s: (b, 0, 0)),
            pl.BlockSpec((1, 1, T), lambda b, s: (b, 0, 0)),
            pl.BlockSpec((1, 1, 1), lambda b, s: (b, 0, 0)),
            pl.BlockSpec((_F, C * _E), lambda b, s: (0, 0)),
            pl.BlockSpec((11, 4), lambda b, s: (0, 0)),
            pl.BlockSpec((12, 4), lambda b, s: (0, 0)),
            pl.BlockSpec((31, 6), lambda b, s: (0, 0)),
            pl.BlockSpec((7, 4), lambda b, s: (0, 0)),
        ],
        out_specs=[
            pl.BlockSpec((1, _SBLK, C * _E), lambda b, s: (b, s, 0)),
            pl.BlockSpec((1, P, C * _E), lambda b, s: (b, 0, 0)),
            pl.BlockSpec((1, P, T), lambda b, s: (b, 0, 0)),
        ],
        out_shape=[
            jax.ShapeDtypeStruct((B, S, C * _E), jnp.float32),
            jax.ShapeDtypeStruct((B, P, C * _E), jnp.float32),
            jax.ShapeDtypeStruct((B, P, T), jnp.float32),
        ],
    )(history_values, stats, history_time_features,
      target_time_features, target_values, tci3, ref_year, m,
      pos_year_table, pos_month_table, pos_day_table, pos_dow_table)

    return out1, out2.reshape(B, P, C, _E), out3


# speedup vs baseline: 5.4932x; 1.0685x over previous
import jax
import jax.numpy as jnp
from jax.experimental import pallas as pl

_YEAR, _MONTH, _DAY, _DOW = 0, 1, 2, 3
_EPS = 0.001
_E = 18
_C = 32
_F = 64

_SBLK = 1024
_LBLK = 1024


def _chunk_stage(y, d, k=None, desc=None):
    n, lanes = y.shape
    g = n // (2 * d)
    xr = y.reshape(g, 2, d, lanes)
    a = xr[:, 0]
    b = xr[:, 1]
    lo = jnp.minimum(a, b)
    hi = jnp.maximum(a, b)
    if desc is None:
        gi = jax.lax.broadcasted_iota(jnp.int32, (g, 1, 1), 0)
        dm = ((gi >> (k - 1 - (d.bit_length() - 1))) & 1) == 1
        na = jnp.where(dm, hi, lo)
        nb = jnp.where(dm, lo, hi)
    elif desc:
        na, nb = hi, lo
    else:
        na, nb = lo, hi
    return jnp.stack([na, nb], axis=1).reshape(n, lanes)


def _sort128(y, final_desc):
    for k in range(1, 8):
        for j in range(k - 1, -1, -1):
            d = 1 << j
            if k == 7:
                y = _chunk_stage(y, d, desc=final_desc)
            else:
                y = _chunk_stage(y, d, k=k)
    return y


def _merge128(y, desc):
    for j in range(6, -1, -1):
        y = _chunk_stage(y, 1 << j, desc=desc)
    return y


def _minmax(a, b, desc):
    lo = jnp.minimum(a, b)
    hi = jnp.maximum(a, b)
    return (hi, lo) if desc else (lo, hi)


def _stats_body(hv_ref, out_ref):
    xb = hv_ref[...]
    x = jnp.concatenate([xb[0], xb[1], xb[2], xb[3]], axis=-1)
    lanes = x.shape[1]
    ch = [x[c * 128:(c + 1) * 128, :] for c in range(8)]
    ch = [_sort128(ch[c], final_desc=bool(c & 1)) for c in range(8)]
    for p in range(4):
        desc = bool(p & 1)
        a, b = _minmax(ch[2 * p], ch[2 * p + 1], desc)
        ch[2 * p] = _merge128(a, desc)
        ch[2 * p + 1] = _merge128(b, desc)
    for g2 in range(2):
        desc = bool(g2 & 1)
        base = g2 * 4
        for t in range(2):
            ch[base + t], ch[base + t + 2] = _minmax(ch[base + t], ch[base + t + 2], desc)
        for t in (0, 2):
            ch[base + t], ch[base + t + 1] = _minmax(ch[base + t], ch[base + t + 1], desc)
        for t in range(4):
            ch[base + t] = _merge128(ch[base + t], desc)
    for t in range(4):
        ch[t], ch[t + 4] = _minmax(ch[t], ch[t + 4], False)
    for t in (0, 1, 4, 5):
        ch[t], ch[t + 2] = _minmax(ch[t], ch[t + 2], False)
    for t in (0, 2, 4, 6):
        ch[t], ch[t + 1] = _minmax(ch[t], ch[t + 1], False)
    ch = [_merge128(c, False) for c in ch]
    med = 0.5 * (ch[3][127:128, :] + ch[4][0:1, :])
    q1 = 0.25 * ch[1][127:128, :] + 0.75 * ch[2][0:1, :]
    q3 = 0.75 * ch[5][127:128, :] + 0.25 * ch[6][0:1, :]
    iqr = (q3 - q1) + _EPS
    for i in range(lanes // _C):
        sl = slice(i * _C, (i + 1) * _C)
        out_ref[i, 0:1, :] = med[:, sl]
        out_ref[i, 1:2, :] = iqr[:, sl]
        out_ref[i, 2:8, :] = jnp.broadcast_to(med[:, sl], (6, _C))


_NOH = 7


def _onehot_feats(tf, ref_year):
    rows = tf.shape[0]
    dy = jnp.clip(ref_year - tf[:, _YEAR:_YEAR + 1], 0, 10)
    vals = jnp.concatenate(
        [jnp.broadcast_to(v, (rows, _NOH)) for v in
         (dy, tf[:, _MONTH:_MONTH + 1], tf[:, _DAY:_DAY + 1],
          tf[:, _DOW:_DOW + 1])], axis=1)
    slot = jax.lax.broadcasted_iota(jnp.int32, (rows, 4 * _NOH), 1) % _NOH
    return jnp.where(vals == slot, 1.0, 0.0)


def _pos_embed_block(tf, ref_year, yt_ref, mt_ref, dt_ref, wt_ref):
    year = tf[:, _YEAR:_YEAR + 1]
    month = tf[:, _MONTH:_MONTH + 1]
    day = tf[:, _DAY:_DAY + 1]
    dow = tf[:, _DOW:_DOW + 1]
    dy = jnp.clip(ref_year - year, 0, 10)

    def lookup(idx, table_ref, rows, width):
        acc = jnp.zeros((idx.shape[0], width), jnp.float32)
        for v in range(rows):
            row = table_ref[v:v + 1, :]
            acc = acc + jnp.where(idx == v, 1.0, 0.0) * row
        return acc

    pe_y = lookup(dy, yt_ref, 11, 4)
    pe_m = lookup(month, mt_ref, 12, 4)
    pe_d = lookup(day, dt_ref, 31, 6)
    pe_w = lookup(dow, wt_ref, 7, 4)
    return pe_y, pe_m, pe_d, pe_w


def _expand_body(hv_ref, st_ref, tf_ref, ttf_ref, tv_ref, tci_ref,
                 ry_ref, m_ref, yt_ref, mt_ref, dt_ref, wt_ref,
                 out1_ref, out2_ref, out3_ref):
    s = pl.program_id(1)
    ref_year = ry_ref[0, 0, 0]
    med = st_ref[0, 0:1, :]
    iqr = st_ref[0, 1:2, :]
    m = m_ref[...]

    hv = hv_ref[0]
    hs = (hv - med) / iqr
    rows = hs.shape[0]
    oh = _onehot_feats(tf_ref[0], ref_year)
    feats = jnp.concatenate(
        [hs, oh, jnp.ones((rows, 1), jnp.float32),
         jnp.zeros((rows, _F - _C - 4 * _NOH - 1), jnp.float32)], axis=-1)
    out1_ref[0] = jax.lax.dot(
        feats, m, precision=jax.lax.Precision.DEFAULT,
        preferred_element_type=jnp.float32)

    @pl.when(s == 0)
    def _targets():
        toh = _onehot_feats(ttf_ref[0], ref_year)
        p = toh.shape[0]
        tfeats = jnp.concatenate(
            [jnp.zeros((p, _C), jnp.float32), toh,
             jnp.zeros((p, _F - _C - 4 * _NOH), jnp.float32)], axis=-1)
        out2_ref[0] = jax.lax.dot(
            tfeats, m, precision=jax.lax.Precision.DEFAULT,
            preferred_element_type=jnp.float32)
        tci = tci_ref[0]
        med_g = jnp.zeros(tci.shape, jnp.float32)
        iqr_g = jnp.zeros(tci.shape, jnp.float32)
        for c in range(_C):
            hit = jnp.where(tci == c, 1.0, 0.0)
            med_g = med_g + hit * med[:, c:c + 1]
            iqr_g = iqr_g + hit * iqr[:, c:c + 1]
        out3_ref[0] = (tv_ref[0] - med_g) / iqr_g


def _build_m(w_expand, b_expand, yt, mt, dt, wt):
    ce = jnp.arange(_C * _E)
    cidx = ce // _E
    rows32 = jnp.arange(_C)[:, None]
    wfull = jnp.tile(w_expand, _C)[None, :]
    m_w = jnp.where(rows32 == cidx[None, :], wfull, 0.0)
    blocks = [m_w]
    for tab, off in ((yt, 0), (mt, 4), (dt, 8), (wt, 14)):
        width = tab.shape[1]
        padded = jnp.pad(tab[:_NOH], ((0, 0), (off, _E - off - width)))
        blocks.append(jnp.tile(padded, (1, _C)))
    blocks.append(jnp.tile(b_expand, _C)[None, :])
    blocks.append(jnp.zeros((_F - _C - 4 * _NOH - 1, _C * _E)))
    return jnp.concatenate(blocks, axis=0).astype(jnp.float32)


@jax.jit
def kernel(history_values, target_values, target_channels_indices,
           history_time_features, target_time_features,
           pos_year_table, pos_month_table, pos_day_table, pos_dow_table,
           W_expand, b_expand):
    B, S, C = history_values.shape
    P, T = target_values.shape[1], target_values.shape[2]

    nlb = (B * C) // _LBLK
    stats = pl.pallas_call(
        _stats_body,
        grid=(nlb,),
        in_specs=[pl.BlockSpec((_LBLK // C, S, C), lambda i: (i, 0, 0))],
        out_specs=pl.BlockSpec((_LBLK // C, 8, C), lambda i: (i, 0, 0)),
        out_shape=jax.ShapeDtypeStruct((B, 8, C), jnp.float32),
    )(history_values)

    m = _build_m(W_expand, b_expand, pos_year_table, pos_month_table,
                 pos_day_table, pos_dow_table)
    ref_year = history_time_features[:, S - 1:S, _YEAR:_YEAR + 1]
    tci3 = target_channels_indices.reshape(B, 1, T)
    nsb = S // _SBLK
    out1, out2, out3 = pl.pallas_call(
        _expand_body,
        grid=(B, nsb),
        in_specs=[
            pl.BlockSpec((1, _SBLK, C), lambda b, s: (b, s, 0)),
            pl.BlockSpec((1, 8, C), lambda b, s: (b, 0, 0)),
            pl.BlockSpec((1, _SBLK, 4), lambda b, s: (b, s, 0)),
            pl.BlockSpec((1, P, 4), lambda b, s: (b, 0, 0)),
            pl.BlockSpec((1, P, T), lambda b, s: (b, 0, 0)),
            pl.BlockSpec((1, 1, T), lambda b, s: (b, 0, 0)),
            pl.BlockSpec((1, 1, 1), lambda b, s: (b, 0, 0)),
            pl.BlockSpec((_F, C * _E), lambda b, s: (0, 0)),
            pl.BlockSpec((11, 4), lambda b, s: (0, 0)),
            pl.BlockSpec((12, 4), lambda b, s: (0, 0)),
            pl.BlockSpec((31, 6), lambda b, s: (0, 0)),
            pl.BlockSpec((7, 4), lambda b, s: (0, 0)),
        ],
        out_specs=[
            pl.BlockSpec((1, _SBLK, C * _E), lambda b, s: (b, s, 0)),
            pl.BlockSpec((1, P, C * _E), lambda b, s: (b, 0, 0)),
            pl.BlockSpec((1, P, T), lambda b, s: (b, 0, 0)),
        ],
        out_shape=[
            jax.ShapeDtypeStruct((B, S, C * _E), jnp.float32),
            jax.ShapeDtypeStruct((B, P, C * _E), jnp.float32),
            jax.ShapeDtypeStruct((B, P, T), jnp.float32),
        ],
    )(history_values, stats, history_time_features,
      target_time_features, target_values, tci3, ref_year, m,
      pos_year_table, pos_month_table, pos_day_table, pos_dow_table)

    return out1, out2.reshape(B, P, C, _E), out3


# speedup vs baseline: 5.6419x; 1.0271x over previous
import jax
import jax.numpy as jnp
from jax.experimental import pallas as pl

_YEAR, _MONTH, _DAY, _DOW = 0, 1, 2, 3
_EPS = 0.001
_E = 18
_C = 32
_F = 64

_SBLK = 1024
_LBLK = 2048


def _chunk_stage(y, d, k=None, desc=None):
    n, lanes = y.shape
    g = n // (2 * d)
    xr = y.reshape(g, 2, d, lanes)
    a = xr[:, 0]
    b = xr[:, 1]
    lo = jnp.minimum(a, b)
    hi = jnp.maximum(a, b)
    if desc is None:
        gi = jax.lax.broadcasted_iota(jnp.int32, (g, 1, 1), 0)
        dm = ((gi >> (k - 1 - (d.bit_length() - 1))) & 1) == 1
        na = jnp.where(dm, hi, lo)
        nb = jnp.where(dm, lo, hi)
    elif desc:
        na, nb = hi, lo
    else:
        na, nb = lo, hi
    return jnp.stack([na, nb], axis=1).reshape(n, lanes)


def _sort128(y, final_desc):
    for k in range(1, 8):
        for j in range(k - 1, -1, -1):
            d = 1 << j
            if k == 7:
                y = _chunk_stage(y, d, desc=final_desc)
            else:
                y = _chunk_stage(y, d, k=k)
    return y


def _merge128(y, desc):
    for j in range(6, -1, -1):
        y = _chunk_stage(y, 1 << j, desc=desc)
    return y


def _minmax(a, b, desc):
    lo = jnp.minimum(a, b)
    hi = jnp.maximum(a, b)
    return (hi, lo) if desc else (lo, hi)


def _stats_body(hv_ref, out_ref):
    xb = hv_ref[...]
    x = jnp.concatenate([xb[0], xb[1], xb[2], xb[3]], axis=-1)
    lanes = x.shape[1]
    ch = [x[c * 128:(c + 1) * 128, :] for c in range(8)]
    ch = [_sort128(ch[c], final_desc=bool(c & 1)) for c in range(8)]
    for p in range(4):
        desc = bool(p & 1)
        a, b = _minmax(ch[2 * p], ch[2 * p + 1], desc)
        ch[2 * p] = _merge128(a, desc)
        ch[2 * p + 1] = _merge128(b, desc)
    for g2 in range(2):
        desc = bool(g2 & 1)
        base = g2 * 4
        for t in range(2):
            ch[base + t], ch[base + t + 2] = _minmax(ch[base + t], ch[base + t + 2], desc)
        for t in (0, 2):
            ch[base + t], ch[base + t + 1] = _minmax(ch[base + t], ch[base + t + 1], desc)
        for t in range(4):
            ch[base + t] = _merge128(ch[base + t], desc)
    for t in range(4):
        ch[t], ch[t + 4] = _minmax(ch[t], ch[t + 4], False)
    for t in (0, 1, 4, 5):
        ch[t], ch[t + 2] = _minmax(ch[t], ch[t + 2], False)
    for t in (0, 2, 4, 6):
        ch[t], ch[t + 1] = _minmax(ch[t], ch[t + 1], False)
    ch = [_merge128(c, False) for c in ch]
    med = 0.5 * (ch[3][127:128, :] + ch[4][0:1, :])
    q1 = 0.25 * ch[1][127:128, :] + 0.75 * ch[2][0:1, :]
    q3 = 0.75 * ch[5][127:128, :] + 0.25 * ch[6][0:1, :]
    iqr = (q3 - q1) + _EPS
    for i in range(lanes // _C):
        sl = slice(i * _C, (i + 1) * _C)
        out_ref[i, 0:1, :] = med[:, sl]
        out_ref[i, 1:2, :] = iqr[:, sl]
        out_ref[i, 2:8, :] = jnp.broadcast_to(med[:, sl], (6, _C))


_NOH = 7


def _onehot_feats(tf, ref_year):
    rows = tf.shape[0]
    dy = jnp.clip(ref_year - tf[:, _YEAR:_YEAR + 1], 0, 10)
    vals = jnp.concatenate(
        [jnp.broadcast_to(v, (rows, _NOH)) for v in
         (dy, tf[:, _MONTH:_MONTH + 1], tf[:, _DAY:_DAY + 1],
          tf[:, _DOW:_DOW + 1])], axis=1)
    slot = jax.lax.broadcasted_iota(jnp.int32, (rows, 4 * _NOH), 1) % _NOH
    return jnp.where(vals == slot, 1.0, 0.0)


def _pos_embed_block(tf, ref_year, yt_ref, mt_ref, dt_ref, wt_ref):
    year = tf[:, _YEAR:_YEAR + 1]
    month = tf[:, _MONTH:_MONTH + 1]
    day = tf[:, _DAY:_DAY + 1]
    dow = tf[:, _DOW:_DOW + 1]
    dy = jnp.clip(ref_year - year, 0, 10)

    def lookup(idx, table_ref, rows, width):
        acc = jnp.zeros((idx.shape[0], width), jnp.float32)
        for v in range(rows):
            row = table_ref[v:v + 1, :]
            acc = acc + jnp.where(idx == v, 1.0, 0.0) * row
        return acc

    pe_y = lookup(dy, yt_ref, 11, 4)
    pe_m = lookup(month, mt_ref, 12, 4)
    pe_d = lookup(day, dt_ref, 31, 6)
    pe_w = lookup(dow, wt_ref, 7, 4)
    return pe_y, pe_m, pe_d, pe_w


def _expand_body(hv_ref, st_ref, tf_ref, ttf_ref, tv_ref, tci_ref,
                 ry_ref, m_ref, yt_ref, mt_ref, dt_ref, wt_ref,
                 out1_ref, out2_ref, out3_ref):
    s = pl.program_id(1)
    ref_year = ry_ref[0, 0, 0]
    med = st_ref[0, 0:1, :]
    iqr = st_ref[0, 1:2, :]
    m = m_ref[...]

    hv = hv_ref[0]
    hs = (hv - med) / iqr
    rows = hs.shape[0]
    oh = _onehot_feats(tf_ref[0], ref_year)
    feats = jnp.concatenate(
        [hs, oh, jnp.ones((rows, 1), jnp.float32),
         jnp.zeros((rows, _F - _C - 4 * _NOH - 1), jnp.float32)], axis=-1)
    out1_ref[0] = jax.lax.dot(
        feats, m, precision=jax.lax.Precision.DEFAULT,
        preferred_element_type=jnp.float32)

    @pl.when(s == 0)
    def _targets():
        toh = _onehot_feats(ttf_ref[0], ref_year)
        p = toh.shape[0]
        tfeats = jnp.concatenate(
            [jnp.zeros((p, _C), jnp.float32), toh,
             jnp.zeros((p, _F - _C - 4 * _NOH), jnp.float32)], axis=-1)
        out2_ref[0] = jax.lax.dot(
            tfeats, m, precision=jax.lax.Precision.DEFAULT,
            preferred_element_type=jnp.float32)
        tci = tci_ref[0]
        med_g = jnp.zeros(tci.shape, jnp.float32)
        iqr_g = jnp.zeros(tci.shape, jnp.float32)
        for c in range(_C):
            hit = jnp.where(tci == c, 1.0, 0.0)
            med_g = med_g + hit * med[:, c:c + 1]
            iqr_g = iqr_g + hit * iqr[:, c:c + 1]
        out3_ref[0] = (tv_ref[0] - med_g) / iqr_g


def _build_m(w_expand, b_expand, yt, mt, dt, wt):
    ce = jnp.arange(_C * _E)
    cidx = ce // _E
    rows32 = jnp.arange(_C)[:, None]
    wfull = jnp.tile(w_expand, _C)[None, :]
    m_w = jnp.where(rows32 == cidx[None, :], wfull, 0.0)
    blocks = [m_w]
    for tab, off in ((yt, 0), (mt, 4), (dt, 8), (wt, 14)):
        width = tab.shape[1]
        padded = jnp.pad(tab[:_NOH], ((0, 0), (off, _E - off - width)))
        blocks.append(jnp.tile(padded, (1, _C)))
    blocks.append(jnp.tile(b_expand, _C)[None, :])
    blocks.append(jnp.zeros((_F - _C - 4 * _NOH - 1, _C * _E)))
    return jnp.concatenate(blocks, axis=0).astype(jnp.float32)


@jax.jit
def kernel(history_values, target_values, target_channels_indices,
           history_time_features, target_time_features,
           pos_year_table, pos_month_table, pos_day_table, pos_dow_table,
           W_expand, b_expand):
    B, S, C = history_values.shape
    P, T = target_values.shape[1], target_values.shape[2]

    nlb = (B * C) // _LBLK
    stats = pl.pallas_call(
        _stats_body,
        grid=(nlb,),
        in_specs=[pl.BlockSpec((_LBLK // C, S, C), lambda i: (i, 0, 0))],
        out_specs=pl.BlockSpec((_LBLK // C, 8, C), lambda i: (i, 0, 0)),
        out_shape=jax.ShapeDtypeStruct((B, 8, C), jnp.float32),
    )(history_values)

    m = _build_m(W_expand, b_expand, pos_year_table, pos_month_table,
                 pos_day_table, pos_dow_table)
    ref_year = history_time_features[:, S - 1:S, _YEAR:_YEAR + 1]
    tci3 = target_channels_indices.reshape(B, 1, T)
    nsb = S // _SBLK
    out1, out2, out3 = pl.pallas_call(
        _expand_body,
        grid=(B, nsb),
        in_specs=[
            pl.BlockSpec((1, _SBLK, C), lambda b, s: (b, s, 0)),
            pl.BlockSpec((1, 8, C), lambda b, s: (b, 0, 0)),
            pl.BlockSpec((1, _SBLK, 4), lambda b, s: (b, s, 0)),
            pl.BlockSpec((1, P, 4), lambda b, s: (b, 0, 0)),
            pl.BlockSpec((1, P, T), lambda b, s: (b, 0, 0)),
            pl.BlockSpec((1, 1, T), lambda b, s: (b, 0, 0)),
            pl.BlockSpec((1, 1, 1), lambda b, s: (b, 0, 0)),
            pl.BlockSpec((_F, C * _E), lambda b, s: (0, 0)),
            pl.BlockSpec((11, 4), lambda b, s: (0, 0)),
            pl.BlockSpec((12, 4), lambda b, s: (0, 0)),
            pl.BlockSpec((31, 6), lambda b, s: (0, 0)),
            pl.BlockSpec((7, 4), lambda b, s: (0, 0)),
        ],
        out_specs=[
            pl.BlockSpec((1, _SBLK, C * _E), lambda b, s: (b, s, 0)),
            pl.BlockSpec((1, P, C * _E), lambda b, s: (b, 0, 0)),
            pl.BlockSpec((1, P, T), lambda b, s: (b, 0, 0)),
        ],
        out_shape=[
            jax.ShapeDtypeStruct((B, S, C * _E), jnp.float32),
            jax.ShapeDtypeStruct((B, P, C * _E), jnp.float32),
            jax.ShapeDtypeStruct((B, P, T), jnp.float32),
        ],
    )(history_values, stats, history_time_features,
      target_time_features, target_values, tci3, ref_year, m,
      pos_year_table, pos_month_table, pos_day_table, pos_dow_table)

    return out1, out2.reshape(B, P, C, _E), out3
